# Initial kernel scaffold; baseline (speedup 1.0000x reference)
import jax
import jax.numpy as jnp
from jax.experimental import pallas as pl


def kernel(node_embeddings, rel_binary, rel_unary, W_in_b, b_in_b, W_out_b, b_out_b, W_in_u, b_in_u, W_out_u, b_out_u, W_in_up, b_in_up, W_out_up, b_out_up):
    raise NotImplementedError("write your pallas kernel here")


# speedup vs baseline: 1.0968x; 1.0968x over previous
import functools

import jax
import jax.numpy as jnp
from jax.experimental import pallas as pl
from jax.experimental.pallas import tpu as pltpu

N_NODES = 10000
D = 128
E_BIN = 640000
E_UNA = 160000
E_ALL = E_BIN + E_UNA

MLP_BLOCK = 1000


def _mish(x):
    sp = jnp.maximum(x, 0.0) + jnp.log1p(jnp.exp(-jnp.abs(x)))
    return x * jnp.tanh(sp)


def _mlp_body(x_ref, wi_ref, bi_ref, wo_ref, bo_ref, o_ref):
    x = x_ref[...]
    h = _mish(jnp.dot(x, wi_ref[0], preferred_element_type=jnp.float32) + bi_ref[0, 0])
    o_ref[...] = x + jnp.dot(h, wo_ref[0], preferred_element_type=jnp.float32) + bo_ref[0, 0]


def _fused_relation_mlp(x_all, wi2, bi2, wo2, bo2):
    rows = x_all.shape[0]
    n_bin_blocks = (E_BIN // 2) // MLP_BLOCK

    def wsel(i):
        return (jnp.where(i < n_bin_blocks, 0, 1), 0, 0)

    def bsel(i):
        return (jnp.where(i < n_bin_blocks, 0, 1), 0, 0)

    return pl.pallas_call(
        _mlp_body,
        grid=(rows // MLP_BLOCK,),
        in_specs=[
            pl.BlockSpec((MLP_BLOCK, 2 * D), lambda i: (i, 0)),
            pl.BlockSpec((1, 2 * D, 2 * D), wsel),
            pl.BlockSpec((1, 1, 2 * D), bsel),
            pl.BlockSpec((1, 2 * D, 2 * D), wsel),
            pl.BlockSpec((1, 1, 2 * D), bsel),
        ],
        out_specs=pl.BlockSpec((MLP_BLOCK, 2 * D), lambda i: (i, 0)),
        out_shape=jax.ShapeDtypeStruct((rows, 2 * D), jnp.float32),
    )(x_all, wi2, bi2, wo2, bo2)


def _update_body(m_ref, e_ref, w1_ref, w2_ref, bi_ref, wo_ref, bo_ref, o_ref):
    h = (jnp.dot(m_ref[...], w1_ref[...], preferred_element_type=jnp.float32)
         + jnp.dot(e_ref[...], w2_ref[...], preferred_element_type=jnp.float32)
         + bi_ref[...])
    o_ref[...] = jnp.dot(_mish(h), wo_ref[...], preferred_element_type=jnp.float32) + bo_ref[...]


def _update_mlp(max_msg, emb, W_in_up, b_in_up, W_out_up, b_out_up):
    blk = 2000
    w1 = W_in_up[:D]
    w2 = W_in_up[D:]
    return pl.pallas_call(
        _update_body,
        grid=(N_NODES // blk,),
        in_specs=[
            pl.BlockSpec((blk, D), lambda i: (i, 0)),
            pl.BlockSpec((blk, D), lambda i: (i, 0)),
            pl.BlockSpec((D, 2 * D), lambda i: (0, 0)),
            pl.BlockSpec((D, 2 * D), lambda i: (0, 0)),
            pl.BlockSpec((2 * D,), lambda i: (0,)),
            pl.BlockSpec((2 * D, D), lambda i: (0, 0)),
            pl.BlockSpec((D,), lambda i: (0,)),
        ],
        out_specs=pl.BlockSpec((blk, D), lambda i: (i, 0)),
        out_shape=jax.ShapeDtypeStruct((N_NODES, D), jnp.float32),
    )(max_msg, emb, w1, w2, b_in_up, W_out_up, b_out_up)


def kernel(node_embeddings, rel_binary, rel_unary,
           W_in_b, b_in_b, W_out_b, b_out_b,
           W_in_u, b_in_u, W_out_u, b_out_u,
           W_in_up, b_in_up, W_out_up, b_out_up):
    idx_all = jnp.concatenate([rel_binary, rel_unary])

    gathered = jnp.take(node_embeddings, idx_all, axis=0)
    x_all = gathered.reshape(E_ALL // 2, 2 * D)

    z = jnp.zeros((D, D), jnp.float32)
    wi_u2 = jnp.block([[W_in_u, z], [z, W_in_u]])
    wo_u2 = jnp.block([[W_out_u, z], [z, W_out_u]])
    wi2 = jnp.stack([W_in_b, wi_u2])
    wo2 = jnp.stack([W_out_b, wo_u2])
    bi2 = jnp.stack([b_in_b, jnp.concatenate([b_in_u, b_in_u])])[:, None, :]
    bo2 = jnp.stack([b_out_b, jnp.concatenate([b_out_u, b_out_u])])[:, None, :]
    msgs = _fused_relation_mlp(x_all, wi2, bi2, wo2, bo2).reshape(E_ALL, D)

    max_msg = jax.ops.segment_max(msgs, idx_all, num_segments=N_NODES)

    return _update_mlp(max_msg, node_embeddings, W_in_up, b_in_up, W_out_up, b_out_up)


# speedup vs baseline: 2.2712x; 2.0708x over previous
import functools

import jax
import jax.numpy as jnp
from jax import lax
from jax.experimental import pallas as pl
from jax.experimental.pallas import tpu as pltpu
from jax.experimental.pallas import tpu_sc as plsc

N_NODES = 10000
D = 128
E_BIN = 640000
E_UNA = 160000
E_ALL = E_BIN + E_UNA

MLP_BLOCK = 1000

NW = 32
GPW = E_ALL // NW
IDX_CHUNK = 1000
SUB = 128
SUB_TAIL = IDX_CHUNK - 7 * SUB

NBUCKET = 32
BUCKET_SZ = 313
BUCKET_MUL = 53602
REGION = 25512
OUT_STRIDE = 320

_NEG_INF = float("-inf")


def _mish(x):
    sp = jnp.maximum(x, 0.0) + jnp.log1p(jnp.exp(-jnp.abs(x)))
    return x * jnp.tanh(sp)


def _gather_body(idx_hbm, table_hbm, out_hbm, idxbuf, rows, sem):
    c = lax.axis_index("c")
    s = lax.axis_index("s")
    wid = s * 2 + c
    base = wid * GPW

    def chunk_body(i, _):
        off = base + i * IDX_CHUNK
        pltpu.sync_copy(idx_hbm.at[pl.ds(off, IDX_CHUNK)], idxbuf)

        def sub_body(m, _):
            pltpu.async_copy(table_hbm.at[idxbuf.at[pl.ds(m * SUB, SUB)]],
                             rows, sem).wait()
            pltpu.sync_copy(rows, out_hbm.at[pl.ds(off + m * SUB, SUB)])
            return 0

        lax.fori_loop(0, 7, sub_body, 0)
        pltpu.async_copy(table_hbm.at[idxbuf.at[pl.ds(7 * SUB, SUB_TAIL)]],
                         rows.at[pl.ds(0, SUB_TAIL)], sem).wait()
        pltpu.sync_copy(rows.at[pl.ds(0, SUB_TAIL)],
                        out_hbm.at[pl.ds(off + 7 * SUB, SUB_TAIL)])
        return 0

    lax.fori_loop(0, GPW // IDX_CHUNK, chunk_body, 0)


def _sc_gather(idx_all, table):
    k = pl.kernel(
        _gather_body,
        out_type=jax.ShapeDtypeStruct((E_ALL, D), jnp.float32),
        mesh=plsc.VectorSubcoreMesh(core_axis_name="c", subcore_axis_name="s"),
        scratch_types=[
            pltpu.VMEM((IDX_CHUNK,), jnp.int32),
            pltpu.VMEM((SUB, D), jnp.float32),
            pltpu.SemaphoreType.DMA,
        ],
    )
    return k(idx_all, table)


def _partition_body(idx_hbm, eid_hbm, ldst_hbm, tab_hbm,
                    idxs, eidbuf, ldstbuf, counters, cursors, startsbuf):
    c = lax.axis_index("c")
    s = lax.axis_index("s")
    wid = s * 2 + c
    base = wid * GPW

    lanes = lax.iota(jnp.int32, 16)
    zeros = jnp.zeros((16,), jnp.int32)
    ones = jnp.ones((16,), jnp.int32)

    pltpu.sync_copy(idx_hbm.at[pl.ds(base, GPW)], idxs.at[pl.ds(0, GPW)])

    idxs[pl.ds(1562 * 16, 16)] = jnp.where(lanes < GPW - 1562 * 16,
                                           idxs[pl.ds(1562 * 16, 16)], zeros)

    for b in range(NBUCKET):
        counters[pl.ds(b * 16, 16)] = zeros

    def zero_body(i, _):
        eidbuf[pl.ds(i * 16, 16)] = zeros
        ldstbuf[pl.ds(i * 16, 16)] = zeros
        return 0

    lax.fori_loop(0, REGION // 16 + 1, zero_body, 0)

    nvec = 1563

    def count_body(i, _):
        m = (i * 16 + lanes) < GPW
        v = idxs[pl.ds(i * 16, 16)]
        bucket = lax.shift_right_logical(v * BUCKET_MUL, 24)
        slot = bucket * 16 + lanes
        plsc.addupdate_scatter(counters, [slot], ones, mask=m)
        return 0

    lax.fori_loop(0, nvec, count_body, 0)

    carry = jnp.int32(0)
    starts = [jnp.int32(0)] * NBUCKET
    ends = [jnp.int32(0)] * NBUCKET
    for b in range(NBUCKET):
        cnts = counters[pl.ds(b * 16, 16)]
        start_al = lax.shift_left(lax.shift_right_logical(carry + 7, 3), 3)
        excl = plsc.cumsum(cnts) - cnts
        cursors[pl.ds(b * 16, 16)] = excl + start_al
        total = jnp.sum(cnts)
        starts[b] = start_al
        ends[b] = start_al + total
        carry = start_al + total

    for j in range(4):
        vals = (starts, ends)[j // 2][(j % 2) * 16:(j % 2) * 16 + 16]
        v = zeros
        for l in range(16):
            v = jnp.where(lanes == l, vals[l], v)
        startsbuf[pl.ds(((j // 2) * 2 + (j % 2)) * 16, 16)] = v

    def place_body(i, _):
        m = (i * 16 + lanes) < GPW
        v = idxs[pl.ds(i * 16, 16)]
        bucket = lax.shift_right_logical(v * BUCKET_MUL, 24)
        local = v - bucket * BUCKET_SZ
        slot = bucket * 16 + lanes
        pos = plsc.load_gather(cursors, [slot])
        eidv = base + i * 16 + lanes
        plsc.store_scatter(eidbuf, [pos], eidv, mask=m)
        plsc.store_scatter(ldstbuf, [pos], local, mask=m)
        plsc.addupdate_scatter(cursors, [slot], ones, mask=m)
        return 0

    lax.fori_loop(0, nvec, place_body, 0)

    pltpu.sync_copy(eidbuf.at[pl.ds(0, REGION)], eid_hbm.at[pl.ds(wid * REGION, REGION)])
    pltpu.sync_copy(ldstbuf.at[pl.ds(0, REGION)], ldst_hbm.at[pl.ds(wid * REGION, REGION)])
    pltpu.sync_copy(startsbuf, tab_hbm.at[pl.ds(wid * 64, 64)])


def _sc_partition(idx_all):
    k = pl.kernel(
        _partition_body,
        out_type=(
            jax.ShapeDtypeStruct((NW * REGION,), jnp.int32),
            jax.ShapeDtypeStruct((NW * REGION,), jnp.int32),
            jax.ShapeDtypeStruct((NW * 64,), jnp.int32),
        ),
        mesh=plsc.VectorSubcoreMesh(core_axis_name="c", subcore_axis_name="s"),
        compiler_params=pltpu.CompilerParams(needs_layout_passes=False),
        scratch_types=[
            pltpu.VMEM((GPW + 16,), jnp.int32),
            pltpu.VMEM((REGION + 16,), jnp.int32),
            pltpu.VMEM((REGION + 16,), jnp.int32),
            pltpu.VMEM((NBUCKET * 16,), jnp.int32),
            pltpu.VMEM((NBUCKET * 16,), jnp.int32),
            pltpu.VMEM((64,), jnp.int32),
        ],
    )
    return k(idx_all)


def _accum_body(msgs_hbm, eid_hbm, ldst_hbm, tab_hbm, out_hbm,
                acc, rowsbuf, eidchunk, ldstchunk, tabbuf, sem):
    c = lax.axis_index("c")
    s = lax.axis_index("s")
    b = s * 2 + c

    neg_inf = jnp.full((16,), _NEG_INF, jnp.float32)

    def init_body(i, _):
        for f in range(8):
            acc[i, pl.ds(f * 16, 16)] = neg_inf
        return 0

    lax.fori_loop(0, OUT_STRIDE, init_body, 0)

    pltpu.sync_copy(tab_hbm, tabbuf)

    lanes = lax.iota(jnp.int32, 16)

    def worker_body(w, _):
        wv = jnp.full((16,), w, jnp.int32)
        st = pl.multiple_of(jnp.max(plsc.load_gather(tabbuf, [wv * 64 + b])), 8)
        en = jnp.max(plsc.load_gather(tabbuf, [wv * 64 + 32 + b]))
        regbase = w * REGION
        nch = lax.shift_right_logical(en - st + 127, 7)

        def chunk_body(j, _):
            cst = st + j * 128
            off = pl.multiple_of(regbase + cst, 8)
            pltpu.sync_copy(eid_hbm.at[pl.ds(off, 128)], eidchunk)
            pltpu.sync_copy(ldst_hbm.at[pl.ds(off, 128)], ldstchunk)
            pltpu.async_copy(msgs_hbm.at[eidchunk], rowsbuf, sem).wait()
            clen = jnp.minimum(jnp.int32(128), en - cst)
            ngroups = lax.shift_right_logical(clen + 15, 4)

            def group_body(g, _):
                rowbase = g * 16
                dvec = ldstchunk[pl.ds(rowbase, 16)]
                dsafe = jnp.where(rowbase + lanes < clen, dvec,
                                  jnp.full((16,), BUCKET_SZ, jnp.int32))
                for l in range(16):
                    d = dsafe[l]
                    r = rowbase + l
                    for f in range(8):
                        cur = acc[d, pl.ds(f * 16, 16)]
                        val = rowsbuf[r, pl.ds(f * 16, 16)]
                        acc[d, pl.ds(f * 16, 16)] = jnp.maximum(cur, val)
                return 0

            lax.fori_loop(0, ngroups, group_body, 0)
            return 0

        lax.fori_loop(0, nch, chunk_body, 0)
        return 0

    lax.fori_loop(0, NW, worker_body, 0)

    pltpu.sync_copy(acc, out_hbm.at[pl.ds(b * OUT_STRIDE, OUT_STRIDE)])


def _sc_accumulate(msgs, eid, ldst, tab):
    k = pl.kernel(
        _accum_body,
        out_type=jax.ShapeDtypeStruct((NW * OUT_STRIDE, D), jnp.float32),
        mesh=plsc.VectorSubcoreMesh(core_axis_name="c", subcore_axis_name="s"),
        compiler_params=pltpu.CompilerParams(needs_layout_passes=False),
        scratch_types=[
            pltpu.VMEM((OUT_STRIDE, D), jnp.float32),
            pltpu.VMEM((128, D), jnp.float32),
            pltpu.VMEM((128,), jnp.int32),
            pltpu.VMEM((128,), jnp.int32),
            pltpu.VMEM((NW * 64,), jnp.int32),
            pltpu.SemaphoreType.DMA,
        ],
    )
    return k(msgs, eid, ldst, tab)


def _mlp_body(x_ref, wi_ref, bi_ref, wo_ref, bo_ref, o_ref):
    x = x_ref[...]
    h = _mish(jnp.dot(x, wi_ref[0], preferred_element_type=jnp.float32) + bi_ref[0, 0])
    o_ref[...] = x + jnp.dot(h, wo_ref[0], preferred_element_type=jnp.float32) + bo_ref[0, 0]


def _fused_relation_mlp(x_all, wi2, bi2, wo2, bo2):
    rows = x_all.shape[0]
    n_bin_blocks = (E_BIN // 2) // MLP_BLOCK

    def wsel(i):
        return (jnp.where(i < n_bin_blocks, 0, 1), 0, 0)

    return pl.pallas_call(
        _mlp_body,
        grid=(rows // MLP_BLOCK,),
        in_specs=[
            pl.BlockSpec((MLP_BLOCK, 2 * D), lambda i: (i, 0)),
            pl.BlockSpec((1, 2 * D, 2 * D), wsel),
            pl.BlockSpec((1, 1, 2 * D), wsel),
            pl.BlockSpec((1, 2 * D, 2 * D), wsel),
            pl.BlockSpec((1, 1, 2 * D), wsel),
        ],
        out_specs=pl.BlockSpec((MLP_BLOCK, 2 * D), lambda i: (i, 0)),
        out_shape=jax.ShapeDtypeStruct((rows, 2 * D), jnp.float32),
    )(x_all, wi2, bi2, wo2, bo2)


def _update_body(m_ref, e_ref, w1_ref, w2_ref, bi_ref, wo_ref, bo_ref, o_ref):
    h = (jnp.dot(m_ref[...], w1_ref[...], preferred_element_type=jnp.float32)
         + jnp.dot(e_ref[...], w2_ref[...], preferred_element_type=jnp.float32)
         + bi_ref[...])
    o_ref[...] = jnp.dot(_mish(h), wo_ref[...], preferred_element_type=jnp.float32) + bo_ref[...]


def _update_mlp(max_msg, emb, W_in_up, b_in_up, W_out_up, b_out_up):
    blk = 2000
    w1 = W_in_up[:D]
    w2 = W_in_up[D:]
    return pl.pallas_call(
        _update_body,
        grid=(N_NODES // blk,),
        in_specs=[
            pl.BlockSpec((blk, D), lambda i: (i, 0)),
            pl.BlockSpec((blk, D), lambda i: (i, 0)),
            pl.BlockSpec((D, 2 * D), lambda i: (0, 0)),
            pl.BlockSpec((D, 2 * D), lambda i: (0, 0)),
            pl.BlockSpec((2 * D,), lambda i: (0,)),
            pl.BlockSpec((2 * D, D), lambda i: (0, 0)),
            pl.BlockSpec((D,), lambda i: (0,)),
        ],
        out_specs=pl.BlockSpec((blk, D), lambda i: (i, 0)),
        out_shape=jax.ShapeDtypeStruct((N_NODES, D), jnp.float32),
    )(max_msg, emb, w1, w2, b_in_up, W_out_up, b_out_up)


def kernel(node_embeddings, rel_binary, rel_unary,
           W_in_b, b_in_b, W_out_b, b_out_b,
           W_in_u, b_in_u, W_out_u, b_out_u,
           W_in_up, b_in_up, W_out_up, b_out_up):
    idx_all = jnp.concatenate([rel_binary, rel_unary])

    gathered = _sc_gather(idx_all, node_embeddings)
    x_all = gathered.reshape(E_ALL // 2, 2 * D)

    z = jnp.zeros((D, D), jnp.float32)
    wi_u2 = jnp.block([[W_in_u, z], [z, W_in_u]])
    wo_u2 = jnp.block([[W_out_u, z], [z, W_out_u]])
    wi2 = jnp.stack([W_in_b, wi_u2])
    wo2 = jnp.stack([W_out_b, wo_u2])
    bi2 = jnp.stack([b_in_b, jnp.concatenate([b_in_u, b_in_u])])[:, None, :]
    bo2 = jnp.stack([b_out_b, jnp.concatenate([b_out_u, b_out_u])])[:, None, :]
    msgs = _fused_relation_mlp(x_all, wi2, bi2, wo2, bo2).reshape(E_ALL, D)

    eid, ldst, tab = _sc_partition(idx_all)
    maxm_padded = _sc_accumulate(msgs, eid, ldst, tab)
    max_msg = maxm_padded.reshape(NW, OUT_STRIDE, D)[:, :BUCKET_SZ].reshape(-1, D)[:N_NODES]

    return _update_mlp(max_msg, node_embeddings, W_in_up, b_in_up, W_out_up, b_out_up)


# speedup vs baseline: 2.5956x; 1.1429x over previous
import functools

import jax
import jax.numpy as jnp
from jax import lax
from jax.experimental import pallas as pl
from jax.experimental.pallas import tpu as pltpu
from jax.experimental.pallas import tpu_sc as plsc

N_NODES = 10000
D = 128
E_BIN = 640000
E_UNA = 160000
E_ALL = E_BIN + E_UNA

MLP_BLOCK = 1000

NW = 32
GPW = E_ALL // NW
IDX_CHUNK = 1000
SUB = 128
SUB_TAIL = IDX_CHUNK - 7 * SUB

NBUCKET = 32
BUCKET_SZ = 313
BUCKET_MUL = 53602
REGION = 25512
OUT_STRIDE = 320

_NEG_INF = float("-inf")


def _mish(x):
    sp = jnp.maximum(x, 0.0) + jnp.log1p(jnp.exp(-jnp.abs(x)))
    return x * jnp.tanh(sp)


def _gather_body(idx_hbm, table_hbm, out_hbm, idxbuf, rows, sem):
    c = lax.axis_index("c")
    s = lax.axis_index("s")
    wid = s * 2 + c
    base = wid * GPW

    def chunk_body(i, _):
        off = base + i * IDX_CHUNK
        pltpu.sync_copy(idx_hbm.at[pl.ds(off, IDX_CHUNK)], idxbuf)

        def sub_body(m, _):
            pltpu.async_copy(table_hbm.at[idxbuf.at[pl.ds(m * SUB, SUB)]],
                             rows, sem).wait()
            pltpu.sync_copy(rows, out_hbm.at[pl.ds(off + m * SUB, SUB)])
            return 0

        lax.fori_loop(0, 7, sub_body, 0)
        pltpu.async_copy(table_hbm.at[idxbuf.at[pl.ds(7 * SUB, SUB_TAIL)]],
                         rows.at[pl.ds(0, SUB_TAIL)], sem).wait()
        pltpu.sync_copy(rows.at[pl.ds(0, SUB_TAIL)],
                        out_hbm.at[pl.ds(off + 7 * SUB, SUB_TAIL)])
        return 0

    lax.fori_loop(0, GPW // IDX_CHUNK, chunk_body, 0)


def _sc_gather(idx_all, table):
    k = pl.kernel(
        _gather_body,
        out_type=jax.ShapeDtypeStruct((E_ALL, D), jnp.float32),
        mesh=plsc.VectorSubcoreMesh(core_axis_name="c", subcore_axis_name="s"),
        scratch_types=[
            pltpu.VMEM((IDX_CHUNK,), jnp.int32),
            pltpu.VMEM((SUB, D), jnp.float32),
            pltpu.SemaphoreType.DMA,
        ],
    )
    return k(idx_all, table)


def _partition_body(idx_hbm, eid_hbm, ldst_hbm, tab_hbm,
                    idxs, eidbuf, ldstbuf, counters, cursors, startsbuf):
    c = lax.axis_index("c")
    s = lax.axis_index("s")
    wid = s * 2 + c
    base = wid * GPW

    lanes = lax.iota(jnp.int32, 16)
    zeros = jnp.zeros((16,), jnp.int32)
    ones = jnp.ones((16,), jnp.int32)

    pltpu.sync_copy(idx_hbm.at[pl.ds(base, GPW)], idxs.at[pl.ds(0, GPW)])

    idxs[pl.ds(1562 * 16, 16)] = jnp.where(lanes < GPW - 1562 * 16,
                                           idxs[pl.ds(1562 * 16, 16)], zeros)

    for b in range(NBUCKET):
        counters[pl.ds(b * 16, 16)] = zeros

    def zero_body(i, _):
        eidbuf[pl.ds(i * 16, 16)] = zeros
        ldstbuf[pl.ds(i * 16, 16)] = zeros
        return 0

    lax.fori_loop(0, REGION // 16 + 1, zero_body, 0)

    nvec = 1563

    def count_body(i, _):
        m = (i * 16 + lanes) < GPW
        v = idxs[pl.ds(i * 16, 16)]
        bucket = lax.shift_right_logical(v * BUCKET_MUL, 24)
        slot = bucket * 16 + lanes
        plsc.addupdate_scatter(counters, [slot], ones, mask=m)
        return 0

    lax.fori_loop(0, nvec, count_body, 0)

    carry = jnp.int32(0)
    starts = [jnp.int32(0)] * NBUCKET
    ends = [jnp.int32(0)] * NBUCKET
    for b in range(NBUCKET):
        cnts = counters[pl.ds(b * 16, 16)]
        start_al = lax.shift_left(lax.shift_right_logical(carry + 7, 3), 3)
        excl = plsc.cumsum(cnts) - cnts
        cursors[pl.ds(b * 16, 16)] = excl + start_al
        total = jnp.sum(cnts)
        starts[b] = start_al
        ends[b] = start_al + total
        carry = start_al + total

    for j in range(4):
        vals = (starts, ends)[j // 2][(j % 2) * 16:(j % 2) * 16 + 16]
        v = zeros
        for l in range(16):
            v = jnp.where(lanes == l, vals[l], v)
        startsbuf[pl.ds(((j // 2) * 2 + (j % 2)) * 16, 16)] = v

    def place_body(i, _):
        m = (i * 16 + lanes) < GPW
        v = idxs[pl.ds(i * 16, 16)]
        bucket = lax.shift_right_logical(v * BUCKET_MUL, 24)
        local = v - bucket * BUCKET_SZ
        slot = bucket * 16 + lanes
        pos = plsc.load_gather(cursors, [slot])
        eidv = base + i * 16 + lanes
        plsc.store_scatter(eidbuf, [pos], eidv, mask=m)
        plsc.store_scatter(ldstbuf, [pos], local, mask=m)
        plsc.addupdate_scatter(cursors, [slot], ones, mask=m)
        return 0

    lax.fori_loop(0, nvec, place_body, 0)

    pltpu.sync_copy(eidbuf.at[pl.ds(0, REGION)], eid_hbm.at[pl.ds(wid * REGION, REGION)])
    pltpu.sync_copy(ldstbuf.at[pl.ds(0, REGION)], ldst_hbm.at[pl.ds(wid * REGION, REGION)])
    pltpu.sync_copy(startsbuf, tab_hbm.at[pl.ds(wid * 64, 64)])


def _sc_partition(idx_all):
    k = pl.kernel(
        _partition_body,
        out_type=(
            jax.ShapeDtypeStruct((NW * REGION,), jnp.int32),
            jax.ShapeDtypeStruct((NW * REGION,), jnp.int32),
            jax.ShapeDtypeStruct((NW * 64,), jnp.int32),
        ),
        mesh=plsc.VectorSubcoreMesh(core_axis_name="c", subcore_axis_name="s"),
        compiler_params=pltpu.CompilerParams(needs_layout_passes=False),
        scratch_types=[
            pltpu.VMEM((GPW + 16,), jnp.int32),
            pltpu.VMEM((REGION + 16,), jnp.int32),
            pltpu.VMEM((REGION + 16,), jnp.int32),
            pltpu.VMEM((NBUCKET * 16,), jnp.int32),
            pltpu.VMEM((NBUCKET * 16,), jnp.int32),
            pltpu.VMEM((64,), jnp.int32),
        ],
    )
    return k(idx_all)


CH = 256


def _accum_body(msgs_hbm, eid_hbm, ldst_hbm, tab_hbm, out_hbm,
                acc, rows0, rows1, ebuf0, ebuf1, lbuf0, lbuf1, tabbuf,
                sg0, sg1, se0, se1):
    c = lax.axis_index("c")
    s = lax.axis_index("s")
    b = s * 2 + c

    neg_inf = jnp.full((16,), _NEG_INF, jnp.float32)
    lanes = lax.iota(jnp.int32, 16)

    def init_body(i, _):
        for f in range(8):
            acc[i, pl.ds(f * 16, 16)] = neg_inf
        return 0

    lax.fori_loop(0, OUT_STRIDE, init_body, 0)

    pltpu.sync_copy(tab_hbm, tabbuf)

    def st_of(w):
        wv = jnp.minimum(jnp.full((16,), w, jnp.int32), 31)
        return pl.multiple_of(jnp.max(plsc.load_gather(tabbuf, [wv * 64 + b])), 8)

    def en_of(w):
        wv = jnp.minimum(jnp.full((16,), w, jnp.int32), 31)
        return jnp.max(plsc.load_gather(tabbuf, [wv * 64 + 32 + b]))

    def count_body(w, t):
        return t + lax.shift_right_logical(
            jnp.maximum(en_of(w) - st_of(w), 0) + CH - 1, 8)

    nchunks = lax.fori_loop(0, NW, count_body, jnp.int32(0))

    def skip_empty(state):
        def cond(st_):
            w_, c_ = st_
            return jnp.logical_and(c_ >= en_of(w_), w_ < NW - 1)

        def body(st_):
            w_, _ = st_
            return (w_ + 1, st_of(w_ + 1))

        return lax.while_loop(cond, body, state)

    def advance(state):
        w_, c_ = state
        return skip_empty((w_, c_ + CH))

    def fire_eid(state, ebuf, lbuf, sem):
        w_, c_ = state
        off = pl.multiple_of(jnp.minimum(w_, 31) * REGION + c_, 8)
        cp1 = pltpu.make_async_copy(eid_hbm.at[pl.ds(off, CH)], ebuf, sem)
        cp2 = pltpu.make_async_copy(ldst_hbm.at[pl.ds(off, CH)], lbuf, sem)
        cp1.start()
        cp2.start()

    def wait_eid(ebuf, lbuf, sem):
        pltpu.make_async_copy(eid_hbm.at[pl.ds(0, CH)], ebuf, sem).wait()
        pltpu.make_async_copy(ldst_hbm.at[pl.ds(0, CH)], lbuf, sem).wait()

    def fire_gather(ebuf, rows, sem):
        pltpu.make_async_copy(msgs_hbm.at[ebuf.at[pl.ds(0, 128)]],
                              rows.at[pl.ds(0, 128)], sem).start()
        pltpu.make_async_copy(msgs_hbm.at[ebuf.at[pl.ds(128, 128)]],
                              rows.at[pl.ds(128, 128)], sem).start()

    def wait_gather(ebuf, rows, sem):
        pltpu.make_async_copy(msgs_hbm.at[ebuf.at[pl.ds(0, 128)]],
                              rows.at[pl.ds(0, 128)], sem).wait()
        pltpu.make_async_copy(msgs_hbm.at[ebuf.at[pl.ds(128, 128)]],
                              rows.at[pl.ds(128, 128)], sem).wait()

    state0 = skip_empty((jnp.int32(0), st_of(jnp.int32(0))))

    @pl.when(nchunks > 0)
    def _prologue():
        fire_eid(state0, ebuf0, lbuf0, se0)
        wait_eid(ebuf0, lbuf0, se0)
        fire_gather(ebuf0, rows0, sg0)
        state1 = advance(state0)

        @pl.when(nchunks > 1)
        def _():
            fire_eid(state1, ebuf1, lbuf1, se1)

    state1 = advance(state0)

    def chunk_loop(j, carry):
        wj, cj, wn, cn = carry
        p = j & 1

        def even_path():
            @pl.when(j + 1 < nchunks)
            def _():
                wait_eid(ebuf1, lbuf1, se1)
                fire_gather(ebuf1, rows1, sg1)

            wait_gather(ebuf0, rows0, sg0)
            _accum_chunk(acc, rows0, lbuf0, wj, cj, lanes)

            @pl.when(j + 2 < nchunks)
            def _():
                fire_eid((wn2, cn2), ebuf0, lbuf0, se0)

        def odd_path():
            @pl.when(j + 1 < nchunks)
            def _():
                wait_eid(ebuf0, lbuf0, se0)
                fire_gather(ebuf0, rows0, sg0)

            wait_gather(ebuf1, rows1, sg1)
            _accum_chunk(acc, rows1, lbuf1, wj, cj, lanes)

            @pl.when(j + 2 < nchunks)
            def _():
                fire_eid((wn2, cn2), ebuf1, lbuf1, se1)

        wn2, cn2 = advance((wn, cn))

        @pl.when(p == 0)
        def _():
            even_path()

        @pl.when(p == 1)
        def _():
            odd_path()

        return (wn, cn, wn2, cn2)

    def _accum_chunk(acc_, rows_, lbuf_, wj, cj, lanes_):
        clen = jnp.minimum(jnp.int32(CH), en_of(wj) - cj)
        ngroups = lax.shift_right_logical(clen + 15, 4)

        def group_body(g, _):
            rowbase = g * 16
            dvec = lbuf_[pl.ds(rowbase, 16)]
            dsafe = jnp.where(rowbase + lanes_ < clen, dvec,
                              jnp.full((16,), BUCKET_SZ, jnp.int32))
            for l in range(16):
                d = dsafe[l]
                r = rowbase + l
                for f in range(8):
                    cur = acc_[d, pl.ds(f * 16, 16)]
                    val = rows_[r, pl.ds(f * 16, 16)]
                    acc_[d, pl.ds(f * 16, 16)] = jnp.maximum(cur, val)
            return 0

        lax.fori_loop(0, ngroups, group_body, 0)

    lax.fori_loop(0, nchunks,
                  chunk_loop,
                  (state0[0], state0[1], state1[0], state1[1]))

    pltpu.sync_copy(acc, out_hbm.at[pl.ds(b * OUT_STRIDE, OUT_STRIDE)])


def _sc_accumulate(msgs, eid, ldst, tab):
    k = pl.kernel(
        _accum_body,
        out_type=jax.ShapeDtypeStruct((NW * OUT_STRIDE, D), jnp.float32),
        mesh=plsc.VectorSubcoreMesh(core_axis_name="c", subcore_axis_name="s"),
        compiler_params=pltpu.CompilerParams(needs_layout_passes=False),
        scratch_types=[
            pltpu.VMEM((OUT_STRIDE, D), jnp.float32),
            pltpu.VMEM((CH, D), jnp.float32),
            pltpu.VMEM((CH, D), jnp.float32),
            pltpu.VMEM((CH,), jnp.int32),
            pltpu.VMEM((CH,), jnp.int32),
            pltpu.VMEM((CH,), jnp.int32),
            pltpu.VMEM((CH,), jnp.int32),
            pltpu.VMEM((NW * 64,), jnp.int32),
            pltpu.SemaphoreType.DMA,
            pltpu.SemaphoreType.DMA,
            pltpu.SemaphoreType.DMA,
            pltpu.SemaphoreType.DMA,
        ],
    )
    return k(msgs, eid, ldst, tab)


def _mlp_body(x_ref, wi_ref, bi_ref, wo_ref, bo_ref, o_ref):
    x = x_ref[...]
    h = _mish(jnp.dot(x.astype(jnp.bfloat16), wi_ref[0],
                      preferred_element_type=jnp.float32) + bi_ref[0, 0])
    o_ref[...] = x + jnp.dot(h.astype(jnp.bfloat16), wo_ref[0],
                             preferred_element_type=jnp.float32) + bo_ref[0, 0]


def _fused_relation_mlp(x_all, wi2, bi2, wo2, bo2):
    rows = x_all.shape[0]
    n_bin_blocks = (E_BIN // 2) // MLP_BLOCK

    def wsel(i):
        return (jnp.where(i < n_bin_blocks, 0, 1), 0, 0)

    return pl.pallas_call(
        _mlp_body,
        grid=(rows // MLP_BLOCK,),
        in_specs=[
            pl.BlockSpec((MLP_BLOCK, 2 * D), lambda i: (i, 0)),
            pl.BlockSpec((1, 2 * D, 2 * D), wsel),
            pl.BlockSpec((1, 1, 2 * D), wsel),
            pl.BlockSpec((1, 2 * D, 2 * D), wsel),
            pl.BlockSpec((1, 1, 2 * D), wsel),
        ],
        out_specs=pl.BlockSpec((MLP_BLOCK, 2 * D), lambda i: (i, 0)),
        out_shape=jax.ShapeDtypeStruct((rows, 2 * D), jnp.float32),
    )(x_all, wi2, bi2, wo2, bo2)


def _update_body(m_ref, e_ref, w1_ref, w2_ref, bi_ref, wo_ref, bo_ref, o_ref):
    h = (jnp.dot(m_ref[...].astype(jnp.bfloat16), w1_ref[...],
                 preferred_element_type=jnp.float32)
         + jnp.dot(e_ref[...].astype(jnp.bfloat16), w2_ref[...],
                   preferred_element_type=jnp.float32)
         + bi_ref[...])
    o_ref[...] = jnp.dot(_mish(h).astype(jnp.bfloat16), wo_ref[...],
                         preferred_element_type=jnp.float32) + bo_ref[...]


def _update_mlp(max_msg, emb, W_in_up, b_in_up, W_out_up, b_out_up):
    blk = 2000
    w1 = W_in_up[:D].astype(jnp.bfloat16)
    w2 = W_in_up[D:].astype(jnp.bfloat16)
    W_out_up = W_out_up.astype(jnp.bfloat16)
    return pl.pallas_call(
        _update_body,
        grid=(N_NODES // blk,),
        in_specs=[
            pl.BlockSpec((blk, D), lambda i: (i, 0)),
            pl.BlockSpec((blk, D), lambda i: (i, 0)),
            pl.BlockSpec((D, 2 * D), lambda i: (0, 0)),
            pl.BlockSpec((D, 2 * D), lambda i: (0, 0)),
            pl.BlockSpec((2 * D,), lambda i: (0,)),
            pl.BlockSpec((2 * D, D), lambda i: (0, 0)),
            pl.BlockSpec((D,), lambda i: (0,)),
        ],
        out_specs=pl.BlockSpec((blk, D), lambda i: (i, 0)),
        out_shape=jax.ShapeDtypeStruct((N_NODES, D), jnp.float32),
    )(max_msg, emb, w1, w2, b_in_up, W_out_up, b_out_up)


def kernel(node_embeddings, rel_binary, rel_unary,
           W_in_b, b_in_b, W_out_b, b_out_b,
           W_in_u, b_in_u, W_out_u, b_out_u,
           W_in_up, b_in_up, W_out_up, b_out_up):
    idx_all = jnp.concatenate([rel_binary, rel_unary])

    gathered = _sc_gather(idx_all, node_embeddings)
    x_all = gathered.reshape(E_ALL // 2, 2 * D)

    z = jnp.zeros((D, D), jnp.float32)
    wi_u2 = jnp.block([[W_in_u, z], [z, W_in_u]])
    wo_u2 = jnp.block([[W_out_u, z], [z, W_out_u]])
    wi2 = jnp.stack([W_in_b, wi_u2]).astype(jnp.bfloat16)
    wo2 = jnp.stack([W_out_b, wo_u2]).astype(jnp.bfloat16)
    bi2 = jnp.stack([b_in_b, jnp.concatenate([b_in_u, b_in_u])])[:, None, :]
    bo2 = jnp.stack([b_out_b, jnp.concatenate([b_out_u, b_out_u])])[:, None, :]
    msgs = _fused_relation_mlp(x_all, wi2, bi2, wo2, bo2).reshape(E_ALL, D)

    eid, ldst, tab = _sc_partition(idx_all)
    maxm_padded = _sc_accumulate(msgs, eid, ldst, tab)
    max_msg = maxm_padded.reshape(NW, OUT_STRIDE, D)[:, :BUCKET_SZ].reshape(-1, D)[:N_NODES]

    return _update_mlp(max_msg, node_embeddings, W_in_up, b_in_up, W_out_up, b_out_up)


# speedup vs baseline: 3.2720x; 1.2606x over previous
import functools

import jax
import jax.numpy as jnp
from jax import lax
from jax.experimental import pallas as pl
from jax.experimental.pallas import tpu as pltpu
from jax.experimental.pallas import tpu_sc as plsc

N_NODES = 10000
D = 128
E_BIN = 640000
E_UNA = 160000
E_ALL = E_BIN + E_UNA

MLP_BLOCK = 1000

NW = 32
GPW = E_ALL // NW
IDX_CHUNK = 1000
SUB = 128
SUB_TAIL = IDX_CHUNK - 7 * SUB

NBUCKET = 32
BUCKET_SZ = 313
BUCKET_MUL = 53602
REGION = 25512
OUT_STRIDE = 320

_NEG_INF = float("-inf")


def _mish(x):
    sp = jnp.maximum(x, 0.0) + jnp.log1p(jnp.exp(-jnp.abs(x)))
    return x * jnp.tanh(sp)


HPW = E_ALL // 2 // 16


def _gather_body(idx_even_hbm, idx_odd_hbm, table_hbm, out_hbm, idxbuf, rows, sem):
    c = lax.axis_index("c")
    s = lax.axis_index("s")
    wid = s * 2 + c

    def run_half(idx_hbm, half, base):
        def chunk_body(i, _):
            off = base + i * IDX_CHUNK
            pltpu.sync_copy(idx_hbm.at[pl.ds(off, IDX_CHUNK)], idxbuf)

            def sub_body(m, _):
                pltpu.async_copy(table_hbm.at[idxbuf.at[pl.ds(m * SUB, SUB)]],
                                 rows, sem).wait()
                pltpu.sync_copy(rows, out_hbm.at[half].at[pl.ds(off + m * SUB, SUB)])
                return 0

            lax.fori_loop(0, 7, sub_body, 0)
            pltpu.async_copy(table_hbm.at[idxbuf.at[pl.ds(7 * SUB, SUB_TAIL)]],
                             rows.at[pl.ds(0, SUB_TAIL)], sem).wait()
            pltpu.sync_copy(rows.at[pl.ds(0, SUB_TAIL)],
                            out_hbm.at[half].at[pl.ds(off + 7 * SUB, SUB_TAIL)])
            return 0

        lax.fori_loop(0, HPW // IDX_CHUNK, chunk_body, 0)

    @pl.when(wid < 16)
    def _():
        run_half(idx_even_hbm, 0, wid * HPW)

    @pl.when(wid >= 16)
    def _():
        run_half(idx_odd_hbm, 1, (wid - 16) * HPW)


def _sc_gather(idx_even, idx_odd, table):
    k = pl.kernel(
        _gather_body,
        out_type=jax.ShapeDtypeStruct((2, E_ALL // 2, D), jnp.float32),
        mesh=plsc.VectorSubcoreMesh(core_axis_name="c", subcore_axis_name="s"),
        scratch_types=[
            pltpu.VMEM((IDX_CHUNK,), jnp.int32),
            pltpu.VMEM((SUB, D), jnp.float32),
            pltpu.SemaphoreType.DMA,
        ],
    )
    return k(idx_even, idx_odd, table)


def _partition_body(idx_hbm, eid_hbm, ldst_hbm, tab_hbm,
                    idxs, eidbuf, ldstbuf, counters, cursors, startsbuf):
    c = lax.axis_index("c")
    s = lax.axis_index("s")
    wid = s * 2 + c
    base = wid * GPW

    lanes = lax.iota(jnp.int32, 16)
    zeros = jnp.zeros((16,), jnp.int32)
    ones = jnp.ones((16,), jnp.int32)

    pltpu.sync_copy(idx_hbm.at[pl.ds(base, GPW)], idxs.at[pl.ds(0, GPW)])

    idxs[pl.ds(1562 * 16, 16)] = jnp.where(lanes < GPW - 1562 * 16,
                                           idxs[pl.ds(1562 * 16, 16)], zeros)

    for b in range(NBUCKET):
        counters[pl.ds(b * 16, 16)] = zeros

    def zero_body(i, _):
        eidbuf[pl.ds(i * 16, 16)] = zeros
        ldstbuf[pl.ds(i * 16, 16)] = zeros
        return 0

    lax.fori_loop(0, REGION // 16 + 1, zero_body, 0)

    nvec = 1563

    def count_body(i, _):
        m = (i * 16 + lanes) < GPW
        v = idxs[pl.ds(i * 16, 16)]
        bucket = lax.shift_right_logical(v * BUCKET_MUL, 24)
        slot = bucket * 16 + lanes
        plsc.addupdate_scatter(counters, [slot], ones, mask=m)
        return 0

    lax.fori_loop(0, nvec, count_body, 0)

    carry = jnp.int32(0)
    starts = [jnp.int32(0)] * NBUCKET
    ends = [jnp.int32(0)] * NBUCKET
    for b in range(NBUCKET):
        cnts = counters[pl.ds(b * 16, 16)]
        start_al = lax.shift_left(lax.shift_right_logical(carry + 7, 3), 3)
        excl = plsc.cumsum(cnts) - cnts
        cursors[pl.ds(b * 16, 16)] = excl + start_al
        total = jnp.sum(cnts)
        starts[b] = start_al
        ends[b] = start_al + total
        carry = start_al + total

    for j in range(4):
        vals = (starts, ends)[j // 2][(j % 2) * 16:(j % 2) * 16 + 16]
        v = zeros
        for l in range(16):
            v = jnp.where(lanes == l, vals[l], v)
        startsbuf[pl.ds(((j // 2) * 2 + (j % 2)) * 16, 16)] = v

    def place_body(i, _):
        m = (i * 16 + lanes) < GPW
        v = idxs[pl.ds(i * 16, 16)]
        bucket = lax.shift_right_logical(v * BUCKET_MUL, 24)
        local = v - bucket * BUCKET_SZ
        slot = bucket * 16 + lanes
        pos = plsc.load_gather(cursors, [slot])
        e = base + i * 16 + lanes
        eidv = (e & 1) * (E_ALL // 2) + lax.shift_right_logical(e, 1)
        plsc.store_scatter(eidbuf, [pos], eidv, mask=m)
        plsc.store_scatter(ldstbuf, [pos], local, mask=m)
        plsc.addupdate_scatter(cursors, [slot], ones, mask=m)
        return 0

    lax.fori_loop(0, nvec, place_body, 0)

    pltpu.sync_copy(eidbuf.at[pl.ds(0, REGION)], eid_hbm.at[pl.ds(wid * REGION, REGION)])
    pltpu.sync_copy(ldstbuf.at[pl.ds(0, REGION)], ldst_hbm.at[pl.ds(wid * REGION, REGION)])
    pltpu.sync_copy(startsbuf, tab_hbm.at[pl.ds(wid * 64, 64)])


def _sc_partition(idx_all):
    k = pl.kernel(
        _partition_body,
        out_type=(
            jax.ShapeDtypeStruct((NW * REGION,), jnp.int32),
            jax.ShapeDtypeStruct((NW * REGION,), jnp.int32),
            jax.ShapeDtypeStruct((NW * 64,), jnp.int32),
        ),
        mesh=plsc.VectorSubcoreMesh(core_axis_name="c", subcore_axis_name="s"),
        compiler_params=pltpu.CompilerParams(needs_layout_passes=False),
        scratch_types=[
            pltpu.VMEM((GPW + 16,), jnp.int32),
            pltpu.VMEM((REGION + 16,), jnp.int32),
            pltpu.VMEM((REGION + 16,), jnp.int32),
            pltpu.VMEM((NBUCKET * 16,), jnp.int32),
            pltpu.VMEM((NBUCKET * 16,), jnp.int32),
            pltpu.VMEM((64,), jnp.int32),
        ],
    )
    return k(idx_all)


CH = 256


def _accum_body(msgs_hbm, eid_hbm, ldst_hbm, tab_hbm, out_hbm,
                acc, rows0, rows1, ebuf0, ebuf1, lbuf0, lbuf1, tabbuf,
                sg0, sg1, se0, se1):
    c = lax.axis_index("c")
    s = lax.axis_index("s")
    b = s * 2 + c

    neg_inf = jnp.full((16,), _NEG_INF, jnp.float32)
    lanes = lax.iota(jnp.int32, 16)

    def init_body(i, _):
        for f in range(8):
            acc[i, pl.ds(f * 16, 16)] = neg_inf
        return 0

    lax.fori_loop(0, OUT_STRIDE, init_body, 0)

    pltpu.sync_copy(tab_hbm, tabbuf)

    def st_of(w):
        wv = jnp.minimum(jnp.full((16,), w, jnp.int32), 31)
        return pl.multiple_of(jnp.max(plsc.load_gather(tabbuf, [wv * 64 + b])), 8)

    def en_of(w):
        wv = jnp.minimum(jnp.full((16,), w, jnp.int32), 31)
        return jnp.max(plsc.load_gather(tabbuf, [wv * 64 + 32 + b]))

    def count_body(w, t):
        return t + lax.shift_right_logical(
            jnp.maximum(en_of(w) - st_of(w), 0) + CH - 1, 8)

    nchunks = lax.fori_loop(0, NW, count_body, jnp.int32(0))

    def skip_empty(state):
        def cond(st_):
            w_, c_ = st_
            return jnp.logical_and(c_ >= en_of(w_), w_ < NW - 1)

        def body(st_):
            w_, _ = st_
            return (w_ + 1, st_of(w_ + 1))

        return lax.while_loop(cond, body, state)

    def advance(state):
        w_, c_ = state
        return skip_empty((w_, c_ + CH))

    def fire_eid(state, ebuf, lbuf, sem):
        w_, c_ = state
        off = pl.multiple_of(jnp.minimum(w_, 31) * REGION + c_, 8)
        cp1 = pltpu.make_async_copy(eid_hbm.at[pl.ds(off, CH)], ebuf, sem)
        cp2 = pltpu.make_async_copy(ldst_hbm.at[pl.ds(off, CH)], lbuf, sem)
        cp1.start()
        cp2.start()

    def wait_eid(ebuf, lbuf, sem):
        pltpu.make_async_copy(eid_hbm.at[pl.ds(0, CH)], ebuf, sem).wait()
        pltpu.make_async_copy(ldst_hbm.at[pl.ds(0, CH)], lbuf, sem).wait()

    def fire_gather(ebuf, rows, sem):
        pltpu.make_async_copy(msgs_hbm.at[ebuf.at[pl.ds(0, 128)]],
                              rows.at[pl.ds(0, 128)], sem).start()
        pltpu.make_async_copy(msgs_hbm.at[ebuf.at[pl.ds(128, 128)]],
                              rows.at[pl.ds(128, 128)], sem).start()

    def wait_gather(ebuf, rows, sem):
        pltpu.make_async_copy(msgs_hbm.at[ebuf.at[pl.ds(0, 128)]],
                              rows.at[pl.ds(0, 128)], sem).wait()
        pltpu.make_async_copy(msgs_hbm.at[ebuf.at[pl.ds(128, 128)]],
                              rows.at[pl.ds(128, 128)], sem).wait()

    state0 = skip_empty((jnp.int32(0), st_of(jnp.int32(0))))

    @pl.when(nchunks > 0)
    def _prologue():
        fire_eid(state0, ebuf0, lbuf0, se0)
        wait_eid(ebuf0, lbuf0, se0)
        fire_gather(ebuf0, rows0, sg0)
        state1 = advance(state0)

        @pl.when(nchunks > 1)
        def _():
            fire_eid(state1, ebuf1, lbuf1, se1)

    state1 = advance(state0)

    def chunk_loop(j, carry):
        wj, cj, wn, cn = carry
        p = j & 1

        def even_path():
            @pl.when(j + 1 < nchunks)
            def _():
                wait_eid(ebuf1, lbuf1, se1)
                fire_gather(ebuf1, rows1, sg1)

            wait_gather(ebuf0, rows0, sg0)
            _accum_chunk(acc, rows0, lbuf0, wj, cj, lanes)

            @pl.when(j + 2 < nchunks)
            def _():
                fire_eid((wn2, cn2), ebuf0, lbuf0, se0)

        def odd_path():
            @pl.when(j + 1 < nchunks)
            def _():
                wait_eid(ebuf0, lbuf0, se0)
                fire_gather(ebuf0, rows0, sg0)

            wait_gather(ebuf1, rows1, sg1)
            _accum_chunk(acc, rows1, lbuf1, wj, cj, lanes)

            @pl.when(j + 2 < nchunks)
            def _():
                fire_eid((wn2, cn2), ebuf1, lbuf1, se1)

        wn2, cn2 = advance((wn, cn))

        @pl.when(p == 0)
        def _():
            even_path()

        @pl.when(p == 1)
        def _():
            odd_path()

        return (wn, cn, wn2, cn2)

    def _accum_chunk(acc_, rows_, lbuf_, wj, cj, lanes_):
        clen = jnp.minimum(jnp.int32(CH), en_of(wj) - cj)
        ngroups = lax.shift_right_logical(clen + 15, 4)

        def group_body(g, _):
            rowbase = g * 16
            dvec = lbuf_[pl.ds(rowbase, 16)]
            dsafe = jnp.where(rowbase + lanes_ < clen, dvec,
                              jnp.full((16,), BUCKET_SZ, jnp.int32))
            for l in range(16):
                d = dsafe[l]
                r = rowbase + l
                for f in range(8):
                    cur = acc_[d, pl.ds(f * 16, 16)]
                    val = rows_[r, pl.ds(f * 16, 16)]
                    acc_[d, pl.ds(f * 16, 16)] = jnp.maximum(cur, val)
            return 0

        lax.fori_loop(0, ngroups, group_body, 0)

    lax.fori_loop(0, nchunks,
                  chunk_loop,
                  (state0[0], state0[1], state1[0], state1[1]))

    pltpu.sync_copy(acc, out_hbm.at[pl.ds(b * OUT_STRIDE, OUT_STRIDE)])


def _sc_accumulate(msgs, eid, ldst, tab):
    k = pl.kernel(
        _accum_body,
        out_type=jax.ShapeDtypeStruct((NW * OUT_STRIDE, D), jnp.float32),
        mesh=plsc.VectorSubcoreMesh(core_axis_name="c", subcore_axis_name="s"),
        compiler_params=pltpu.CompilerParams(needs_layout_passes=False),
        scratch_types=[
            pltpu.VMEM((OUT_STRIDE, D), jnp.float32),
            pltpu.VMEM((CH, D), jnp.float32),
            pltpu.VMEM((CH, D), jnp.float32),
            pltpu.VMEM((CH,), jnp.int32),
            pltpu.VMEM((CH,), jnp.int32),
            pltpu.VMEM((CH,), jnp.int32),
            pltpu.VMEM((CH,), jnp.int32),
            pltpu.VMEM((NW * 64,), jnp.int32),
            pltpu.SemaphoreType.DMA,
            pltpu.SemaphoreType.DMA,
            pltpu.SemaphoreType.DMA,
            pltpu.SemaphoreType.DMA,
        ],
    )
    return k(msgs, eid, ldst, tab)


def _mlp_body(x_ref, wi_ref, bi_ref, wo_ref, bo_ref, o_ref):
    x3 = x_ref[...]
    xe = x3[0]
    xo = x3[1]
    wi = wi_ref[0]
    wo = wo_ref[0]
    h = _mish(jnp.dot(xe.astype(jnp.bfloat16), wi[:D],
                      preferred_element_type=jnp.float32)
              + jnp.dot(xo.astype(jnp.bfloat16), wi[D:],
                        preferred_element_type=jnp.float32)
              + bi_ref[0, 0])
    hb = h.astype(jnp.bfloat16)
    bo = bo_ref[0, 0]
    o_ref[0] = xe + jnp.dot(hb, wo[:, :D], preferred_element_type=jnp.float32) + bo[:D]
    o_ref[1] = xo + jnp.dot(hb, wo[:, D:], preferred_element_type=jnp.float32) + bo[D:]


def _fused_relation_mlp(x3, wi2, bi2, wo2, bo2):
    rows = x3.shape[1]
    n_bin_blocks = (E_BIN // 2) // MLP_BLOCK

    def wsel(i):
        return (jnp.where(i < n_bin_blocks, 0, 1), 0, 0)

    return pl.pallas_call(
        _mlp_body,
        grid=(rows // MLP_BLOCK,),
        in_specs=[
            pl.BlockSpec((2, MLP_BLOCK, D), lambda i: (0, i, 0)),
            pl.BlockSpec((1, 2 * D, 2 * D), wsel),
            pl.BlockSpec((1, 1, 2 * D), wsel),
            pl.BlockSpec((1, 2 * D, 2 * D), wsel),
            pl.BlockSpec((1, 1, 2 * D), wsel),
        ],
        out_specs=pl.BlockSpec((2, MLP_BLOCK, D), lambda i: (0, i, 0)),
        out_shape=jax.ShapeDtypeStruct((2, rows, D), jnp.float32),
    )(x3, wi2, bi2, wo2, bo2)


def _update_body(m_ref, e_ref, w1_ref, w2_ref, bi_ref, wo_ref, bo_ref, o_ref):
    h = (jnp.dot(m_ref[...].astype(jnp.bfloat16), w1_ref[...],
                 preferred_element_type=jnp.float32)
         + jnp.dot(e_ref[...].astype(jnp.bfloat16), w2_ref[...],
                   preferred_element_type=jnp.float32)
         + bi_ref[...])
    o_ref[...] = jnp.dot(_mish(h).astype(jnp.bfloat16), wo_ref[...],
                         preferred_element_type=jnp.float32) + bo_ref[...]


def _update_mlp(max_msg, emb, W_in_up, b_in_up, W_out_up, b_out_up):
    blk = 2000
    w1 = W_in_up[:D].astype(jnp.bfloat16)
    w2 = W_in_up[D:].astype(jnp.bfloat16)
    W_out_up = W_out_up.astype(jnp.bfloat16)
    return pl.pallas_call(
        _update_body,
        grid=(N_NODES // blk,),
        in_specs=[
            pl.BlockSpec((blk, D), lambda i: (i, 0)),
            pl.BlockSpec((blk, D), lambda i: (i, 0)),
            pl.BlockSpec((D, 2 * D), lambda i: (0, 0)),
            pl.BlockSpec((D, 2 * D), lambda i: (0, 0)),
            pl.BlockSpec((2 * D,), lambda i: (0,)),
            pl.BlockSpec((2 * D, D), lambda i: (0, 0)),
            pl.BlockSpec((D,), lambda i: (0,)),
        ],
        out_specs=pl.BlockSpec((blk, D), lambda i: (i, 0)),
        out_shape=jax.ShapeDtypeStruct((N_NODES, D), jnp.float32),
    )(max_msg, emb, w1, w2, b_in_up, W_out_up, b_out_up)


def kernel(node_embeddings, rel_binary, rel_unary,
           W_in_b, b_in_b, W_out_b, b_out_b,
           W_in_u, b_in_u, W_out_u, b_out_u,
           W_in_up, b_in_up, W_out_up, b_out_up):
    idx_all = jnp.concatenate([rel_binary, rel_unary])
    idx_even = idx_all[0::2]
    idx_odd = idx_all[1::2]

    gathered = _sc_gather(idx_even, idx_odd, node_embeddings)

    z = jnp.zeros((D, D), jnp.float32)
    wi_u2 = jnp.block([[W_in_u, z], [z, W_in_u]])
    wo_u2 = jnp.block([[W_out_u, z], [z, W_out_u]])
    wi2 = jnp.stack([W_in_b, wi_u2]).astype(jnp.bfloat16)
    wo2 = jnp.stack([W_out_b, wo_u2]).astype(jnp.bfloat16)
    bi2 = jnp.stack([b_in_b, jnp.concatenate([b_in_u, b_in_u])])[:, None, :]
    bo2 = jnp.stack([b_out_b, jnp.concatenate([b_out_u, b_out_u])])[:, None, :]
    msgs = _fused_relation_mlp(gathered, wi2, bi2, wo2, bo2).reshape(E_ALL, D)

    eid, ldst, tab = _sc_partition(idx_all)
    maxm_padded = _sc_accumulate(msgs, eid, ldst, tab)
    max_msg = maxm_padded.reshape(NW, OUT_STRIDE, D)[:, :BUCKET_SZ].reshape(-1, D)[:N_NODES]

    return _update_mlp(max_msg, node_embeddings, W_in_up, b_in_up, W_out_up, b_out_up)


# speedup vs baseline: 4.0711x; 1.2442x over previous
import functools

import jax
import jax.numpy as jnp
from jax import lax
from jax.experimental import pallas as pl
from jax.experimental.pallas import tpu as pltpu
from jax.experimental.pallas import tpu_sc as plsc

N_NODES = 10000
D = 128
E_BIN = 640000
E_UNA = 160000
E_ALL = E_BIN + E_UNA

MLP_BLOCK = 1000

NW = 32
GPW = E_ALL // NW
IDX_CHUNK = 1000
SUB = 128
SUB_TAIL = IDX_CHUNK - 7 * SUB

NBUCKET = 32
BUCKET_SZ = 313
BUCKET_MUL = 53602
REGION = 25512
OUT_STRIDE = 320

_NEG_INF = float("-inf")


def _mish(x):
    sp = jnp.maximum(x, 0.0) + jnp.log1p(jnp.exp(-jnp.abs(x)))
    return x * jnp.tanh(sp)


HPW = E_ALL // 2 // 16


def _gather_body(idx_even_hbm, idx_odd_hbm, table_hbm, out_hbm, idxbuf, rows, sem):
    c = lax.axis_index("c")
    s = lax.axis_index("s")
    wid = s * 2 + c

    def run_half(idx_hbm, half, base):
        def chunk_body(i, _):
            off = base + i * IDX_CHUNK
            pltpu.sync_copy(idx_hbm.at[pl.ds(off, IDX_CHUNK)], idxbuf)

            def sub_body(m, _):
                pltpu.async_copy(table_hbm.at[idxbuf.at[pl.ds(m * SUB, SUB)]],
                                 rows, sem).wait()
                pltpu.sync_copy(rows, out_hbm.at[half].at[pl.ds(off + m * SUB, SUB)])
                return 0

            lax.fori_loop(0, 7, sub_body, 0)
            pltpu.async_copy(table_hbm.at[idxbuf.at[pl.ds(7 * SUB, SUB_TAIL)]],
                             rows.at[pl.ds(0, SUB_TAIL)], sem).wait()
            pltpu.sync_copy(rows.at[pl.ds(0, SUB_TAIL)],
                            out_hbm.at[half].at[pl.ds(off + 7 * SUB, SUB_TAIL)])
            return 0

        lax.fori_loop(0, HPW // IDX_CHUNK, chunk_body, 0)

    @pl.when(wid < 16)
    def _():
        run_half(idx_even_hbm, 0, wid * HPW)

    @pl.when(wid >= 16)
    def _():
        run_half(idx_odd_hbm, 1, (wid - 16) * HPW)


def _sc_gather(idx_even, idx_odd, table):
    k = pl.kernel(
        _gather_body,
        out_type=jax.ShapeDtypeStruct((2, E_ALL // 2, D), jnp.float32),
        mesh=plsc.VectorSubcoreMesh(core_axis_name="c", subcore_axis_name="s"),
        scratch_types=[
            pltpu.VMEM((IDX_CHUNK,), jnp.int32),
            pltpu.VMEM((SUB, D), jnp.float32),
            pltpu.SemaphoreType.DMA,
        ],
    )
    return k(idx_even, idx_odd, table)


def _partition_body(idx_hbm, eid_hbm, ldst_hbm, tab_hbm,
                    idxs, eidbuf, ldstbuf, counters, cursors, startsbuf):
    c = lax.axis_index("c")
    s = lax.axis_index("s")
    wid = s * 2 + c
    base = wid * GPW

    lanes = lax.iota(jnp.int32, 16)
    zeros = jnp.zeros((16,), jnp.int32)
    ones = jnp.ones((16,), jnp.int32)

    pltpu.sync_copy(idx_hbm.at[pl.ds(base, GPW)], idxs.at[pl.ds(0, GPW)])

    idxs[pl.ds(1562 * 16, 16)] = jnp.where(lanes < GPW - 1562 * 16,
                                           idxs[pl.ds(1562 * 16, 16)], zeros)

    for b in range(NBUCKET):
        counters[pl.ds(b * 16, 16)] = zeros

    def zero_body(i, _):
        eidbuf[pl.ds(i * 16, 16)] = zeros
        ldstbuf[pl.ds(i * 16, 16)] = zeros
        return 0

    lax.fori_loop(0, REGION // 16 + 1, zero_body, 0)

    nvec = 1563

    def count_body(i, _):
        m = (i * 16 + lanes) < GPW
        v = idxs[pl.ds(i * 16, 16)]
        bucket = lax.shift_right_logical(v * BUCKET_MUL, 24)
        slot = bucket * 16 + lanes
        plsc.addupdate_scatter(counters, [slot], ones, mask=m)
        return 0

    lax.fori_loop(0, nvec, count_body, 0)

    carry = jnp.int32(0)
    starts = [jnp.int32(0)] * NBUCKET
    ends = [jnp.int32(0)] * NBUCKET
    for b in range(NBUCKET):
        cnts = counters[pl.ds(b * 16, 16)]
        start_al = lax.shift_left(lax.shift_right_logical(carry + 7, 3), 3)
        excl = plsc.cumsum(cnts) - cnts
        cursors[pl.ds(b * 16, 16)] = excl + start_al
        total = jnp.sum(cnts)
        starts[b] = start_al
        ends[b] = start_al + total
        carry = start_al + total

    for j in range(4):
        vals = (starts, ends)[j // 2][(j % 2) * 16:(j % 2) * 16 + 16]
        v = zeros
        for l in range(16):
            v = jnp.where(lanes == l, vals[l], v)
        startsbuf[pl.ds(((j // 2) * 2 + (j % 2)) * 16, 16)] = v

    def place_body(i, _):
        m = (i * 16 + lanes) < GPW
        v = idxs[pl.ds(i * 16, 16)]
        bucket = lax.shift_right_logical(v * BUCKET_MUL, 24)
        local = v - bucket * BUCKET_SZ
        slot = bucket * 16 + lanes
        pos = plsc.load_gather(cursors, [slot])
        e = base + i * 16 + lanes
        eidv = (e & 1) * (E_ALL // 2) + lax.shift_right_logical(e, 1)
        plsc.store_scatter(eidbuf, [pos], eidv, mask=m)
        plsc.store_scatter(ldstbuf, [pos], local, mask=m)
        plsc.addupdate_scatter(cursors, [slot], ones, mask=m)
        return 0

    lax.fori_loop(0, nvec, place_body, 0)

    pltpu.sync_copy(eidbuf.at[pl.ds(0, REGION)], eid_hbm.at[pl.ds(wid * REGION, REGION)])
    pltpu.sync_copy(ldstbuf.at[pl.ds(0, REGION)], ldst_hbm.at[pl.ds(wid * REGION, REGION)])
    pltpu.sync_copy(startsbuf, tab_hbm.at[pl.ds(wid * 64, 64)])


def _sc_partition(idx_all):
    k = pl.kernel(
        _partition_body,
        out_type=(
            jax.ShapeDtypeStruct((NW * REGION,), jnp.int32),
            jax.ShapeDtypeStruct((NW * REGION,), jnp.int32),
            jax.ShapeDtypeStruct((NW * 64,), jnp.int32),
        ),
        mesh=plsc.VectorSubcoreMesh(core_axis_name="c", subcore_axis_name="s"),
        compiler_params=pltpu.CompilerParams(needs_layout_passes=False),
        scratch_types=[
            pltpu.VMEM((GPW + 16,), jnp.int32),
            pltpu.VMEM((REGION + 16,), jnp.int32),
            pltpu.VMEM((REGION + 16,), jnp.int32),
            pltpu.VMEM((NBUCKET * 16,), jnp.int32),
            pltpu.VMEM((NBUCKET * 16,), jnp.int32),
            pltpu.VMEM((64,), jnp.int32),
        ],
    )
    return k(idx_all)


CH = 256


def _accum_body(msgs_hbm, eid_hbm, ldst_hbm, tab_hbm, out_hbm,
                acc, rows0, rows1, ebuf0, ebuf1, lbuf0, lbuf1, tabbuf,
                sg0, sg1, se0, se1):
    c = lax.axis_index("c")
    s = lax.axis_index("s")
    b = s * 2 + c

    neg_inf = jnp.full((16,), _NEG_INF, jnp.float32)
    lanes = lax.iota(jnp.int32, 16)

    def init_body(i, _):
        for f in range(8):
            acc[i, pl.ds(f * 16, 16)] = neg_inf
        return 0

    lax.fori_loop(0, OUT_STRIDE, init_body, 0)

    pltpu.sync_copy(tab_hbm, tabbuf)

    def st_of(w):
        wv = jnp.minimum(jnp.full((16,), w, jnp.int32), 31)
        return pl.multiple_of(jnp.max(plsc.load_gather(tabbuf, [wv * 64 + b])), 8)

    def en_of(w):
        wv = jnp.minimum(jnp.full((16,), w, jnp.int32), 31)
        return jnp.max(plsc.load_gather(tabbuf, [wv * 64 + 32 + b]))

    def count_body(w, t):
        return t + lax.shift_right_logical(
            jnp.maximum(en_of(w) - st_of(w), 0) + CH - 1, 8)

    nchunks = lax.fori_loop(0, NW, count_body, jnp.int32(0))

    def skip_empty(state):
        def cond(st_):
            w_, c_ = st_
            return jnp.logical_and(c_ >= en_of(w_), w_ < NW - 1)

        def body(st_):
            w_, _ = st_
            return (w_ + 1, st_of(w_ + 1))

        return lax.while_loop(cond, body, state)

    def advance(state):
        w_, c_ = state
        return skip_empty((w_, c_ + CH))

    def fire_eid(state, ebuf, lbuf, sem):
        w_, c_ = state
        off = pl.multiple_of(jnp.minimum(w_, 31) * REGION + c_, 8)
        cp1 = pltpu.make_async_copy(eid_hbm.at[pl.ds(off, CH)], ebuf, sem)
        cp2 = pltpu.make_async_copy(ldst_hbm.at[pl.ds(off, CH)], lbuf, sem)
        cp1.start()
        cp2.start()

    def wait_eid(ebuf, lbuf, sem):
        pltpu.make_async_copy(eid_hbm.at[pl.ds(0, CH)], ebuf, sem).wait()
        pltpu.make_async_copy(ldst_hbm.at[pl.ds(0, CH)], lbuf, sem).wait()

    def fire_gather(ebuf, rows, sem):
        pltpu.make_async_copy(msgs_hbm.at[ebuf.at[pl.ds(0, 128)]],
                              rows.at[pl.ds(0, 128)], sem).start()
        pltpu.make_async_copy(msgs_hbm.at[ebuf.at[pl.ds(128, 128)]],
                              rows.at[pl.ds(128, 128)], sem).start()

    def wait_gather(ebuf, rows, sem):
        pltpu.make_async_copy(msgs_hbm.at[ebuf.at[pl.ds(0, 128)]],
                              rows.at[pl.ds(0, 128)], sem).wait()
        pltpu.make_async_copy(msgs_hbm.at[ebuf.at[pl.ds(128, 128)]],
                              rows.at[pl.ds(128, 128)], sem).wait()

    state0 = skip_empty((jnp.int32(0), st_of(jnp.int32(0))))

    @pl.when(nchunks > 0)
    def _prologue():
        fire_eid(state0, ebuf0, lbuf0, se0)
        wait_eid(ebuf0, lbuf0, se0)
        fire_gather(ebuf0, rows0, sg0)
        state1 = advance(state0)

        @pl.when(nchunks > 1)
        def _():
            fire_eid(state1, ebuf1, lbuf1, se1)

    state1 = advance(state0)

    def chunk_loop(j, carry):
        wj, cj, wn, cn = carry
        p = j & 1

        def even_path():
            @pl.when(j + 1 < nchunks)
            def _():
                wait_eid(ebuf1, lbuf1, se1)
                fire_gather(ebuf1, rows1, sg1)

            wait_gather(ebuf0, rows0, sg0)
            _accum_chunk(acc, rows0, lbuf0, wj, cj, lanes)

            @pl.when(j + 2 < nchunks)
            def _():
                fire_eid((wn2, cn2), ebuf0, lbuf0, se0)

        def odd_path():
            @pl.when(j + 1 < nchunks)
            def _():
                wait_eid(ebuf0, lbuf0, se0)
                fire_gather(ebuf0, rows0, sg0)

            wait_gather(ebuf1, rows1, sg1)
            _accum_chunk(acc, rows1, lbuf1, wj, cj, lanes)

            @pl.when(j + 2 < nchunks)
            def _():
                fire_eid((wn2, cn2), ebuf1, lbuf1, se1)

        wn2, cn2 = advance((wn, cn))

        @pl.when(p == 0)
        def _():
            even_path()

        @pl.when(p == 1)
        def _():
            odd_path()

        return (wn, cn, wn2, cn2)

    def _accum_chunk(acc_, rows_, lbuf_, wj, cj, lanes_):
        clen = jnp.minimum(jnp.int32(CH), en_of(wj) - cj)
        ngroups = lax.shift_right_logical(clen + 15, 4)

        def group_body(g, _):
            rowbase = g * 16
            dvec = lbuf_[pl.ds(rowbase, 16)]
            dsafe = jnp.where(rowbase + lanes_ < clen, dvec,
                              jnp.full((16,), BUCKET_SZ, jnp.int32))
            for l in range(16):
                d = dsafe[l]
                r = rowbase + l
                vals = []
                for f in range(8):
                    cur = acc_[d, pl.ds(f * 16, 16)]
                    val = rows_[r, pl.ds(f * 16, 16)]
                    vals.append(jnp.maximum(cur, val))
                for f in range(8):
                    acc_[d, pl.ds(f * 16, 16)] = vals[f]
            return 0

        lax.fori_loop(0, ngroups, group_body, 0)

    lax.fori_loop(0, nchunks,
                  chunk_loop,
                  (state0[0], state0[1], state1[0], state1[1]))

    pltpu.sync_copy(acc, out_hbm.at[pl.ds(b * OUT_STRIDE, OUT_STRIDE)])


def _sc_accumulate(msgs, eid, ldst, tab):
    k = pl.kernel(
        _accum_body,
        out_type=jax.ShapeDtypeStruct((NW * OUT_STRIDE, D), jnp.float32),
        mesh=plsc.VectorSubcoreMesh(core_axis_name="c", subcore_axis_name="s"),
        compiler_params=pltpu.CompilerParams(needs_layout_passes=False),
        scratch_types=[
            pltpu.VMEM((OUT_STRIDE, D), jnp.float32),
            pltpu.VMEM((CH, D), jnp.float32),
            pltpu.VMEM((CH, D), jnp.float32),
            pltpu.VMEM((CH,), jnp.int32),
            pltpu.VMEM((CH,), jnp.int32),
            pltpu.VMEM((CH,), jnp.int32),
            pltpu.VMEM((CH,), jnp.int32),
            pltpu.VMEM((NW * 64,), jnp.int32),
            pltpu.SemaphoreType.DMA,
            pltpu.SemaphoreType.DMA,
            pltpu.SemaphoreType.DMA,
            pltpu.SemaphoreType.DMA,
        ],
    )
    return k(msgs, eid, ldst, tab)


def _mlp_body(x_ref, wi_ref, bi_ref, wo_ref, bo_ref, o_ref):
    x3 = x_ref[...]
    xe = x3[0]
    xo = x3[1]
    wi = wi_ref[0]
    wo = wo_ref[0]
    h = _mish(jnp.dot(xe.astype(jnp.bfloat16), wi[:D],
                      preferred_element_type=jnp.float32)
              + jnp.dot(xo.astype(jnp.bfloat16), wi[D:],
                        preferred_element_type=jnp.float32)
              + bi_ref[0, 0])
    hb = h.astype(jnp.bfloat16)
    bo = bo_ref[0, 0]
    o_ref[0] = xe + jnp.dot(hb, wo[:, :D], preferred_element_type=jnp.float32) + bo[:D]
    o_ref[1] = xo + jnp.dot(hb, wo[:, D:], preferred_element_type=jnp.float32) + bo[D:]


def _fused_relation_mlp(x3, wi2, bi2, wo2, bo2):
    rows = x3.shape[1]
    n_bin_blocks = (E_BIN // 2) // MLP_BLOCK

    def wsel(i):
        return (jnp.where(i < n_bin_blocks, 0, 1), 0, 0)

    return pl.pallas_call(
        _mlp_body,
        grid=(rows // MLP_BLOCK,),
        in_specs=[
            pl.BlockSpec((2, MLP_BLOCK, D), lambda i: (0, i, 0)),
            pl.BlockSpec((1, 2 * D, 2 * D), wsel),
            pl.BlockSpec((1, 1, 2 * D), wsel),
            pl.BlockSpec((1, 2 * D, 2 * D), wsel),
            pl.BlockSpec((1, 1, 2 * D), wsel),
        ],
        out_specs=pl.BlockSpec((2, MLP_BLOCK, D), lambda i: (0, i, 0)),
        out_shape=jax.ShapeDtypeStruct((2, rows, D), jnp.float32),
    )(x3, wi2, bi2, wo2, bo2)


def _update_body(m_ref, e_ref, w1_ref, w2_ref, bi_ref, wo_ref, bo_ref, o_ref):
    h = (jnp.dot(m_ref[...].astype(jnp.bfloat16), w1_ref[...],
                 preferred_element_type=jnp.float32)
         + jnp.dot(e_ref[...].astype(jnp.bfloat16), w2_ref[...],
                   preferred_element_type=jnp.float32)
         + bi_ref[...])
    o_ref[...] = jnp.dot(_mish(h).astype(jnp.bfloat16), wo_ref[...],
                         preferred_element_type=jnp.float32) + bo_ref[...]


def _update_mlp(max_msg, emb, W_in_up, b_in_up, W_out_up, b_out_up):
    blk = 2000
    w1 = W_in_up[:D].astype(jnp.bfloat16)
    w2 = W_in_up[D:].astype(jnp.bfloat16)
    W_out_up = W_out_up.astype(jnp.bfloat16)
    return pl.pallas_call(
        _update_body,
        grid=(N_NODES // blk,),
        in_specs=[
            pl.BlockSpec((blk, D), lambda i: (i, 0)),
            pl.BlockSpec((blk, D), lambda i: (i, 0)),
            pl.BlockSpec((D, 2 * D), lambda i: (0, 0)),
            pl.BlockSpec((D, 2 * D), lambda i: (0, 0)),
            pl.BlockSpec((2 * D,), lambda i: (0,)),
            pl.BlockSpec((2 * D, D), lambda i: (0, 0)),
            pl.BlockSpec((D,), lambda i: (0,)),
        ],
        out_specs=pl.BlockSpec((blk, D), lambda i: (i, 0)),
        out_shape=jax.ShapeDtypeStruct((N_NODES, D), jnp.float32),
    )(max_msg, emb, w1, w2, b_in_up, W_out_up, b_out_up)


def kernel(node_embeddings, rel_binary, rel_unary,
           W_in_b, b_in_b, W_out_b, b_out_b,
           W_in_u, b_in_u, W_out_u, b_out_u,
           W_in_up, b_in_up, W_out_up, b_out_up):
    idx_all = jnp.concatenate([rel_binary, rel_unary])
    idx_even = idx_all[0::2]
    idx_odd = idx_all[1::2]

    gathered = _sc_gather(idx_even, idx_odd, node_embeddings)

    z = jnp.zeros((D, D), jnp.float32)
    wi_u2 = jnp.block([[W_in_u, z], [z, W_in_u]])
    wo_u2 = jnp.block([[W_out_u, z], [z, W_out_u]])
    wi2 = jnp.stack([W_in_b, wi_u2]).astype(jnp.bfloat16)
    wo2 = jnp.stack([W_out_b, wo_u2]).astype(jnp.bfloat16)
    bi2 = jnp.stack([b_in_b, jnp.concatenate([b_in_u, b_in_u])])[:, None, :]
    bo2 = jnp.stack([b_out_b, jnp.concatenate([b_out_u, b_out_u])])[:, None, :]
    msgs = _fused_relation_mlp(gathered, wi2, bi2, wo2, bo2).reshape(E_ALL, D)

    eid, ldst, tab = _sc_partition(idx_all)
    maxm_padded = _sc_accumulate(msgs, eid, ldst, tab)
    max_msg = maxm_padded.reshape(NW, OUT_STRIDE, D)[:, :BUCKET_SZ].reshape(-1, D)[:N_NODES]

    return _update_mlp(max_msg, node_embeddings, W_in_up, b_in_up, W_out_up, b_out_up)


# speedup vs baseline: 4.6055x; 1.1313x over previous
import functools

import jax
import jax.numpy as jnp
from jax import lax
from jax.experimental import pallas as pl
from jax.experimental.pallas import tpu as pltpu
from jax.experimental.pallas import tpu_sc as plsc

N_NODES = 10000
D = 128
E_BIN = 640000
E_UNA = 160000
E_ALL = E_BIN + E_UNA

MLP_BLOCK = 2000

NW = 32
GPW = E_ALL // NW
IDX_CHUNK = 1000
SUB = 128
SUB_TAIL = IDX_CHUNK - 7 * SUB

NBUCKET = 32
BUCKET_SZ = 313
BUCKET_MUL = 53602
REGION = 25512
OUT_STRIDE = 320

_NEG_INF = float("-inf")


def _mish(x):
    sp = jnp.maximum(x, 0.0) + jnp.log1p(jnp.exp(-jnp.abs(x)))
    return x * jnp.tanh(sp)


HPW = E_ALL // 2 // 16


def _gather_body(idx_all_hbm, table_hbm, out_hbm, stage, idxbuf, rows, sem):
    c = lax.axis_index("c")
    s = lax.axis_index("s")
    wid = s * 2 + c
    lanes = lax.iota(jnp.int32, 16)

    def run_half(par, half, base):
        def chunk_body(i, _):
            off = base + i * IDX_CHUNK
            pltpu.sync_copy(idx_all_hbm.at[pl.ds(2 * off, 2 * IDX_CHUNK)],
                            stage.at[pl.ds(0, 2 * IDX_CHUNK)])

            def deint_body(k, _):
                v = plsc.load_gather(stage, [k * 32 + lanes * 2 + par])
                idxbuf[pl.ds(k * 16, 16)] = v
                return 0

            lax.fori_loop(0, (IDX_CHUNK + 15) // 16, deint_body, 0)

            def sub_body(m, _):
                pltpu.async_copy(table_hbm.at[idxbuf.at[pl.ds(m * SUB, SUB)]],
                                 rows, sem).wait()
                pltpu.sync_copy(rows, out_hbm.at[half].at[pl.ds(off + m * SUB, SUB)])
                return 0

            lax.fori_loop(0, 7, sub_body, 0)
            pltpu.async_copy(table_hbm.at[idxbuf.at[pl.ds(7 * SUB, SUB_TAIL)]],
                             rows.at[pl.ds(0, SUB_TAIL)], sem).wait()
            pltpu.sync_copy(rows.at[pl.ds(0, SUB_TAIL)],
                            out_hbm.at[half].at[pl.ds(off + 7 * SUB, SUB_TAIL)])
            return 0

        lax.fori_loop(0, HPW // IDX_CHUNK, chunk_body, 0)

    @pl.when(wid < 16)
    def _():
        run_half(0, 0, wid * HPW)

    @pl.when(wid >= 16)
    def _():
        run_half(1, 1, (wid - 16) * HPW)


def _sc_gather(idx_all, table):
    k = pl.kernel(
        _gather_body,
        out_type=jax.ShapeDtypeStruct((2, E_ALL // 2, D), jnp.float32),
        mesh=plsc.VectorSubcoreMesh(core_axis_name="c", subcore_axis_name="s"),
        compiler_params=pltpu.CompilerParams(needs_layout_passes=False),
        scratch_types=[
            pltpu.VMEM((2 * IDX_CHUNK + 16,), jnp.int32),
            pltpu.VMEM((IDX_CHUNK + 16,), jnp.int32),
            pltpu.VMEM((SUB, D), jnp.float32),
            pltpu.SemaphoreType.DMA,
        ],
    )
    return k(idx_all, table)


def _partition_body(idx_hbm, eid_hbm, ldst_hbm, tab_hbm,
                    idxs, eidbuf, ldstbuf, counters, cursors, startsbuf):
    c = lax.axis_index("c")
    s = lax.axis_index("s")
    wid = s * 2 + c
    base = wid * GPW

    lanes = lax.iota(jnp.int32, 16)
    zeros = jnp.zeros((16,), jnp.int32)
    ones = jnp.ones((16,), jnp.int32)

    pltpu.sync_copy(idx_hbm.at[pl.ds(base, GPW)], idxs.at[pl.ds(0, GPW)])

    idxs[pl.ds(1562 * 16, 16)] = jnp.where(lanes < GPW - 1562 * 16,
                                           idxs[pl.ds(1562 * 16, 16)], zeros)

    for b in range(NBUCKET):
        counters[pl.ds(b * 16, 16)] = zeros

    def zero_body(i, _):
        eidbuf[pl.ds(i * 16, 16)] = zeros
        ldstbuf[pl.ds(i * 16, 16)] = zeros
        return 0

    lax.fori_loop(0, REGION // 16 + 1, zero_body, 0)

    nvec = 1563

    def count_body(i, _):
        m = (i * 16 + lanes) < GPW
        v = idxs[pl.ds(i * 16, 16)]
        bucket = lax.shift_right_logical(v * BUCKET_MUL, 24)
        slot = bucket * 16 + lanes
        plsc.addupdate_scatter(counters, [slot], ones, mask=m)
        return 0

    lax.fori_loop(0, nvec, count_body, 0)

    carry = jnp.int32(0)
    starts = [jnp.int32(0)] * NBUCKET
    ends = [jnp.int32(0)] * NBUCKET
    for b in range(NBUCKET):
        cnts = counters[pl.ds(b * 16, 16)]
        start_al = lax.shift_left(lax.shift_right_logical(carry + 7, 3), 3)
        excl = plsc.cumsum(cnts) - cnts
        cursors[pl.ds(b * 16, 16)] = excl + start_al
        total = jnp.sum(cnts)
        starts[b] = start_al
        ends[b] = start_al + total
        carry = start_al + total

    for j in range(4):
        vals = (starts, ends)[j // 2][(j % 2) * 16:(j % 2) * 16 + 16]
        v = zeros
        for l in range(16):
            v = jnp.where(lanes == l, vals[l], v)
        startsbuf[pl.ds(((j // 2) * 2 + (j % 2)) * 16, 16)] = v

    def place_body(i, _):
        m = (i * 16 + lanes) < GPW
        v = idxs[pl.ds(i * 16, 16)]
        bucket = lax.shift_right_logical(v * BUCKET_MUL, 24)
        local = v - bucket * BUCKET_SZ
        slot = bucket * 16 + lanes
        pos = plsc.load_gather(cursors, [slot])
        e = base + i * 16 + lanes
        eidv = (e & 1) * (E_ALL // 2) + lax.shift_right_logical(e, 1)
        plsc.store_scatter(eidbuf, [pos], eidv, mask=m)
        plsc.store_scatter(ldstbuf, [pos], local, mask=m)
        plsc.addupdate_scatter(cursors, [slot], ones, mask=m)
        return 0

    lax.fori_loop(0, nvec, place_body, 0)

    pltpu.sync_copy(eidbuf.at[pl.ds(0, REGION)], eid_hbm.at[pl.ds(wid * REGION, REGION)])
    pltpu.sync_copy(ldstbuf.at[pl.ds(0, REGION)], ldst_hbm.at[pl.ds(wid * REGION, REGION)])
    pltpu.sync_copy(startsbuf, tab_hbm.at[pl.ds(wid * 64, 64)])


def _sc_partition(idx_all):
    k = pl.kernel(
        _partition_body,
        out_type=(
            jax.ShapeDtypeStruct((NW * REGION,), jnp.int32),
            jax.ShapeDtypeStruct((NW * REGION,), jnp.int32),
            jax.ShapeDtypeStruct((NW * 64,), jnp.int32),
        ),
        mesh=plsc.VectorSubcoreMesh(core_axis_name="c", subcore_axis_name="s"),
        compiler_params=pltpu.CompilerParams(needs_layout_passes=False),
        scratch_types=[
            pltpu.VMEM((GPW + 16,), jnp.int32),
            pltpu.VMEM((REGION + 16,), jnp.int32),
            pltpu.VMEM((REGION + 16,), jnp.int32),
            pltpu.VMEM((NBUCKET * 16,), jnp.int32),
            pltpu.VMEM((NBUCKET * 16,), jnp.int32),
            pltpu.VMEM((64,), jnp.int32),
        ],
    )
    return k(idx_all)


CH = 256


def _accum_body(msgs_hbm, eid_hbm, ldst_hbm, tab_hbm, out_hbm,
                acc, rows0, rows1, ebuf0, ebuf1, lbuf0, lbuf1, tabbuf,
                sg0, sg1, se0, se1):
    c = lax.axis_index("c")
    s = lax.axis_index("s")
    b = s * 2 + c

    neg_inf = jnp.full((16,), _NEG_INF, jnp.float32)
    lanes = lax.iota(jnp.int32, 16)

    def init_body(i, _):
        for f in range(8):
            acc[i, pl.ds(f * 16, 16)] = neg_inf
        return 0

    lax.fori_loop(0, OUT_STRIDE, init_body, 0)

    pltpu.sync_copy(tab_hbm, tabbuf)

    def st_of(w):
        wv = jnp.minimum(jnp.full((16,), w, jnp.int32), 31)
        return pl.multiple_of(jnp.max(plsc.load_gather(tabbuf, [wv * 64 + b])), 8)

    def en_of(w):
        wv = jnp.minimum(jnp.full((16,), w, jnp.int32), 31)
        return jnp.max(plsc.load_gather(tabbuf, [wv * 64 + 32 + b]))

    def count_body(w, t):
        return t + lax.shift_right_logical(
            jnp.maximum(en_of(w) - st_of(w), 0) + CH - 1, 8)

    nchunks = lax.fori_loop(0, NW, count_body, jnp.int32(0))

    def skip_empty(state):
        def cond(st_):
            w_, c_ = st_
            return jnp.logical_and(c_ >= en_of(w_), w_ < NW - 1)

        def body(st_):
            w_, _ = st_
            return (w_ + 1, st_of(w_ + 1))

        return lax.while_loop(cond, body, state)

    def advance(state):
        w_, c_ = state
        return skip_empty((w_, c_ + CH))

    def fire_eid(state, ebuf, lbuf, sem):
        w_, c_ = state
        off = pl.multiple_of(jnp.minimum(w_, 31) * REGION + c_, 8)
        cp1 = pltpu.make_async_copy(eid_hbm.at[pl.ds(off, CH)], ebuf, sem)
        cp2 = pltpu.make_async_copy(ldst_hbm.at[pl.ds(off, CH)], lbuf, sem)
        cp1.start()
        cp2.start()

    def wait_eid(ebuf, lbuf, sem):
        pltpu.make_async_copy(eid_hbm.at[pl.ds(0, CH)], ebuf, sem).wait()
        pltpu.make_async_copy(ldst_hbm.at[pl.ds(0, CH)], lbuf, sem).wait()

    def fire_gather(ebuf, rows, sem):
        pltpu.make_async_copy(msgs_hbm.at[ebuf.at[pl.ds(0, 128)]],
                              rows.at[pl.ds(0, 128)], sem).start()
        pltpu.make_async_copy(msgs_hbm.at[ebuf.at[pl.ds(128, 128)]],
                              rows.at[pl.ds(128, 128)], sem).start()

    def wait_gather(ebuf, rows, sem):
        pltpu.make_async_copy(msgs_hbm.at[ebuf.at[pl.ds(0, 128)]],
                              rows.at[pl.ds(0, 128)], sem).wait()
        pltpu.make_async_copy(msgs_hbm.at[ebuf.at[pl.ds(128, 128)]],
                              rows.at[pl.ds(128, 128)], sem).wait()

    state0 = skip_empty((jnp.int32(0), st_of(jnp.int32(0))))

    @pl.when(nchunks > 0)
    def _prologue():
        fire_eid(state0, ebuf0, lbuf0, se0)
        wait_eid(ebuf0, lbuf0, se0)
        fire_gather(ebuf0, rows0, sg0)
        state1 = advance(state0)

        @pl.when(nchunks > 1)
        def _():
            fire_eid(state1, ebuf1, lbuf1, se1)

    state1 = advance(state0)

    def chunk_loop(j, carry):
        wj, cj, wn, cn = carry
        p = j & 1

        def even_path():
            @pl.when(j + 1 < nchunks)
            def _():
                wait_eid(ebuf1, lbuf1, se1)
                fire_gather(ebuf1, rows1, sg1)

            wait_gather(ebuf0, rows0, sg0)
            _accum_chunk(acc, rows0, lbuf0, wj, cj, lanes)

            @pl.when(j + 2 < nchunks)
            def _():
                fire_eid((wn2, cn2), ebuf0, lbuf0, se0)

        def odd_path():
            @pl.when(j + 1 < nchunks)
            def _():
                wait_eid(ebuf0, lbuf0, se0)
                fire_gather(ebuf0, rows0, sg0)

            wait_gather(ebuf1, rows1, sg1)
            _accum_chunk(acc, rows1, lbuf1, wj, cj, lanes)

            @pl.when(j + 2 < nchunks)
            def _():
                fire_eid((wn2, cn2), ebuf1, lbuf1, se1)

        wn2, cn2 = advance((wn, cn))

        @pl.when(p == 0)
        def _():
            even_path()

        @pl.when(p == 1)
        def _():
            odd_path()

        return (wn, cn, wn2, cn2)

    def _accum_chunk(acc_, rows_, lbuf_, wj, cj, lanes_):
        clen = jnp.minimum(jnp.int32(CH), en_of(wj) - cj)
        ngroups = lax.shift_right_logical(clen + 15, 4)

        def group_body(g, _):
            rowbase = g * 16
            dvec = lbuf_[pl.ds(rowbase, 16)]
            dsafe = jnp.where(rowbase + lanes_ < clen, dvec,
                              jnp.full((16,), BUCKET_SZ, jnp.int32))
            for l in range(16):
                d = dsafe[l]
                r = rowbase + l
                vals = []
                for f in range(8):
                    cur = acc_[d, pl.ds(f * 16, 16)]
                    val = rows_[r, pl.ds(f * 16, 16)]
                    vals.append(jnp.maximum(cur, val))
                for f in range(8):
                    acc_[d, pl.ds(f * 16, 16)] = vals[f]
            return 0

        lax.fori_loop(0, ngroups, group_body, 0)

    lax.fori_loop(0, nchunks,
                  chunk_loop,
                  (state0[0], state0[1], state1[0], state1[1]))

    pltpu.sync_copy(acc, out_hbm.at[pl.ds(b * OUT_STRIDE, OUT_STRIDE)])


def _sc_accumulate(msgs, eid, ldst, tab):
    k = pl.kernel(
        _accum_body,
        out_type=jax.ShapeDtypeStruct((NW * OUT_STRIDE, D), jnp.float32),
        mesh=plsc.VectorSubcoreMesh(core_axis_name="c", subcore_axis_name="s"),
        compiler_params=pltpu.CompilerParams(needs_layout_passes=False),
        scratch_types=[
            pltpu.VMEM((OUT_STRIDE, D), jnp.float32),
            pltpu.VMEM((CH, D), jnp.float32),
            pltpu.VMEM((CH, D), jnp.float32),
            pltpu.VMEM((CH,), jnp.int32),
            pltpu.VMEM((CH,), jnp.int32),
            pltpu.VMEM((CH,), jnp.int32),
            pltpu.VMEM((CH,), jnp.int32),
            pltpu.VMEM((NW * 64,), jnp.int32),
            pltpu.SemaphoreType.DMA,
            pltpu.SemaphoreType.DMA,
            pltpu.SemaphoreType.DMA,
            pltpu.SemaphoreType.DMA,
        ],
    )
    return k(msgs, eid, ldst, tab)


def _mlp_body(x_ref, wi_ref, bi_ref, wo_ref, bo_ref, o_ref):
    x3 = x_ref[...]
    xe = x3[0]
    xo = x3[1]
    wi = wi_ref[0]
    wo = wo_ref[0]
    h = _mish(jnp.dot(xe.astype(jnp.bfloat16), wi[:D],
                      preferred_element_type=jnp.float32)
              + jnp.dot(xo.astype(jnp.bfloat16), wi[D:],
                        preferred_element_type=jnp.float32)
              + bi_ref[0, 0])
    hb = h.astype(jnp.bfloat16)
    bo = bo_ref[0, 0]
    o_ref[0] = xe + jnp.dot(hb, wo[:, :D], preferred_element_type=jnp.float32) + bo[:D]
    o_ref[1] = xo + jnp.dot(hb, wo[:, D:], preferred_element_type=jnp.float32) + bo[D:]


def _fused_relation_mlp(x3, wi2, bi2, wo2, bo2):
    rows = x3.shape[1]
    n_bin_blocks = (E_BIN // 2) // MLP_BLOCK

    def wsel(i):
        return (jnp.where(i < n_bin_blocks, 0, 1), 0, 0)

    return pl.pallas_call(
        _mlp_body,
        grid=(rows // MLP_BLOCK,),
        in_specs=[
            pl.BlockSpec((2, MLP_BLOCK, D), lambda i: (0, i, 0)),
            pl.BlockSpec((1, 2 * D, 2 * D), wsel),
            pl.BlockSpec((1, 1, 2 * D), wsel),
            pl.BlockSpec((1, 2 * D, 2 * D), wsel),
            pl.BlockSpec((1, 1, 2 * D), wsel),
        ],
        out_specs=pl.BlockSpec((2, MLP_BLOCK, D), lambda i: (0, i, 0)),
        out_shape=jax.ShapeDtypeStruct((2, rows, D), jnp.float32),
    )(x3, wi2, bi2, wo2, bo2)


def _update_body(m_ref, e_ref, w1_ref, w2_ref, bi_ref, wo_ref, bo_ref, o_ref):
    h = (jnp.dot(m_ref[...].astype(jnp.bfloat16), w1_ref[...],
                 preferred_element_type=jnp.float32)
         + jnp.dot(e_ref[...].astype(jnp.bfloat16), w2_ref[...],
                   preferred_element_type=jnp.float32)
         + bi_ref[...])
    o_ref[...] = jnp.dot(_mish(h).astype(jnp.bfloat16), wo_ref[...],
                         preferred_element_type=jnp.float32) + bo_ref[...]


def _update_mlp(max_msg, emb, W_in_up, b_in_up, W_out_up, b_out_up):
    blk = 2000
    w1 = W_in_up[:D].astype(jnp.bfloat16)
    w2 = W_in_up[D:].astype(jnp.bfloat16)
    W_out_up = W_out_up.astype(jnp.bfloat16)
    return pl.pallas_call(
        _update_body,
        grid=(N_NODES // blk,),
        in_specs=[
            pl.BlockSpec((blk, D), lambda i: (i, 0)),
            pl.BlockSpec((blk, D), lambda i: (i, 0)),
            pl.BlockSpec((D, 2 * D), lambda i: (0, 0)),
            pl.BlockSpec((D, 2 * D), lambda i: (0, 0)),
            pl.BlockSpec((2 * D,), lambda i: (0,)),
            pl.BlockSpec((2 * D, D), lambda i: (0, 0)),
            pl.BlockSpec((D,), lambda i: (0,)),
        ],
        out_specs=pl.BlockSpec((blk, D), lambda i: (i, 0)),
        out_shape=jax.ShapeDtypeStruct((N_NODES, D), jnp.float32),
    )(max_msg, emb, w1, w2, b_in_up, W_out_up, b_out_up)


def kernel(node_embeddings, rel_binary, rel_unary,
           W_in_b, b_in_b, W_out_b, b_out_b,
           W_in_u, b_in_u, W_out_u, b_out_u,
           W_in_up, b_in_up, W_out_up, b_out_up):
    idx_all = jnp.concatenate([rel_binary, rel_unary])

    gathered = _sc_gather(idx_all, node_embeddings)

    z = jnp.zeros((D, D), jnp.float32)
    wi_u2 = jnp.block([[W_in_u, z], [z, W_in_u]])
    wo_u2 = jnp.block([[W_out_u, z], [z, W_out_u]])
    wi2 = jnp.stack([W_in_b, wi_u2]).astype(jnp.bfloat16)
    wo2 = jnp.stack([W_out_b, wo_u2]).astype(jnp.bfloat16)
    bi2 = jnp.stack([b_in_b, jnp.concatenate([b_in_u, b_in_u])])[:, None, :]
    bo2 = jnp.stack([b_out_b, jnp.concatenate([b_out_u, b_out_u])])[:, None, :]
    msgs = _fused_relation_mlp(gathered, wi2, bi2, wo2, bo2).reshape(E_ALL, D)

    eid, ldst, tab = _sc_partition(idx_all)
    maxm_padded = _sc_accumulate(msgs, eid, ldst, tab)
    max_msg = maxm_padded.reshape(NW, OUT_STRIDE, D)[:, :BUCKET_SZ].reshape(-1, D)[:N_NODES]

    return _update_mlp(max_msg, node_embeddings, W_in_up, b_in_up, W_out_up, b_out_up)


# speedup vs baseline: 4.6835x; 1.0169x over previous
import functools

import jax
import jax.numpy as jnp
from jax import lax
from jax.experimental import pallas as pl
from jax.experimental.pallas import tpu as pltpu
from jax.experimental.pallas import tpu_sc as plsc

N_NODES = 10000
D = 128
E_BIN = 640000
E_UNA = 160000
E_ALL = E_BIN + E_UNA

MLP_BLOCK = 4000

NW = 32
GPW = E_ALL // NW
IDX_CHUNK = 1000
SUB = 128
SUB_TAIL = IDX_CHUNK - 7 * SUB

NBUCKET = 32
BUCKET_SZ = 313
BUCKET_MUL = 53602
REGION = 25512
OUT_STRIDE = 320

_NEG_INF = float("-inf")


def _mish(x):
    sp = jnp.maximum(x, 0.0) + jnp.log1p(jnp.exp(-jnp.abs(x)))
    return x * jnp.tanh(sp)


HPW = E_ALL // 2 // 16


def _gather_body(idx_all_hbm, table_hbm, out_hbm, stage, idxbuf, rows, sem):
    c = lax.axis_index("c")
    s = lax.axis_index("s")
    wid = s * 2 + c
    lanes = lax.iota(jnp.int32, 16)

    def run_half(par, half, base):
        def chunk_body(i, _):
            off = base + i * IDX_CHUNK
            pltpu.sync_copy(idx_all_hbm.at[pl.ds(2 * off, 2 * IDX_CHUNK)],
                            stage.at[pl.ds(0, 2 * IDX_CHUNK)])

            def deint_body(k, _):
                v = plsc.load_gather(stage, [k * 32 + lanes * 2 + par])
                idxbuf[pl.ds(k * 16, 16)] = v
                return 0

            lax.fori_loop(0, (IDX_CHUNK + 15) // 16, deint_body, 0)

            def sub_body(m, _):
                pltpu.async_copy(table_hbm.at[idxbuf.at[pl.ds(m * SUB, SUB)]],
                                 rows, sem).wait()
                pltpu.sync_copy(rows, out_hbm.at[half].at[pl.ds(off + m * SUB, SUB)])
                return 0

            lax.fori_loop(0, 7, sub_body, 0)
            pltpu.async_copy(table_hbm.at[idxbuf.at[pl.ds(7 * SUB, SUB_TAIL)]],
                             rows.at[pl.ds(0, SUB_TAIL)], sem).wait()
            pltpu.sync_copy(rows.at[pl.ds(0, SUB_TAIL)],
                            out_hbm.at[half].at[pl.ds(off + 7 * SUB, SUB_TAIL)])
            return 0

        lax.fori_loop(0, HPW // IDX_CHUNK, chunk_body, 0)

    @pl.when(wid < 16)
    def _():
        run_half(0, 0, wid * HPW)

    @pl.when(wid >= 16)
    def _():
        run_half(1, 1, (wid - 16) * HPW)


def _sc_gather(idx_all, table):
    k = pl.kernel(
        _gather_body,
        out_type=jax.ShapeDtypeStruct((2, E_ALL // 2, D), jnp.float32),
        mesh=plsc.VectorSubcoreMesh(core_axis_name="c", subcore_axis_name="s"),
        compiler_params=pltpu.CompilerParams(needs_layout_passes=False),
        scratch_types=[
            pltpu.VMEM((2 * IDX_CHUNK + 16,), jnp.int32),
            pltpu.VMEM((IDX_CHUNK + 16,), jnp.int32),
            pltpu.VMEM((SUB, D), jnp.float32),
            pltpu.SemaphoreType.DMA,
        ],
    )
    return k(idx_all, table)


def _partition_body(idx_hbm, eid_hbm, ldst_hbm, tab_hbm,
                    idxs, eidbuf, ldstbuf, counters, cursors, startsbuf):
    c = lax.axis_index("c")
    s = lax.axis_index("s")
    wid = s * 2 + c
    base = wid * GPW

    lanes = lax.iota(jnp.int32, 16)
    zeros = jnp.zeros((16,), jnp.int32)
    ones = jnp.ones((16,), jnp.int32)

    pltpu.sync_copy(idx_hbm.at[pl.ds(base, GPW)], idxs.at[pl.ds(0, GPW)])

    idxs[pl.ds(1562 * 16, 16)] = jnp.where(lanes < GPW - 1562 * 16,
                                           idxs[pl.ds(1562 * 16, 16)], zeros)

    for b in range(NBUCKET):
        counters[pl.ds(b * 16, 16)] = zeros

    def zero_body(i, _):
        eidbuf[pl.ds(i * 16, 16)] = zeros
        ldstbuf[pl.ds(i * 16, 16)] = zeros
        return 0

    lax.fori_loop(0, REGION // 16 + 1, zero_body, 0)

    nvec = 1563

    def count_body(i, _):
        m = (i * 16 + lanes) < GPW
        v = idxs[pl.ds(i * 16, 16)]
        bucket = lax.shift_right_logical(v * BUCKET_MUL, 24)
        slot = bucket * 16 + lanes
        plsc.addupdate_scatter(counters, [slot], ones, mask=m)
        return 0

    lax.fori_loop(0, nvec, count_body, 0)

    carry = jnp.int32(0)
    starts = [jnp.int32(0)] * NBUCKET
    ends = [jnp.int32(0)] * NBUCKET
    for b in range(NBUCKET):
        cnts = counters[pl.ds(b * 16, 16)]
        start_al = lax.shift_left(lax.shift_right_logical(carry + 7, 3), 3)
        excl = plsc.cumsum(cnts) - cnts
        cursors[pl.ds(b * 16, 16)] = excl + start_al
        total = jnp.sum(cnts)
        starts[b] = start_al
        ends[b] = start_al + total
        carry = start_al + total

    for j in range(4):
        vals = (starts, ends)[j // 2][(j % 2) * 16:(j % 2) * 16 + 16]
        v = zeros
        for l in range(16):
            v = jnp.where(lanes == l, vals[l], v)
        startsbuf[pl.ds(((j // 2) * 2 + (j % 2)) * 16, 16)] = v

    def place_body(i, _):
        m = (i * 16 + lanes) < GPW
        v = idxs[pl.ds(i * 16, 16)]
        bucket = lax.shift_right_logical(v * BUCKET_MUL, 24)
        local = v - bucket * BUCKET_SZ
        slot = bucket * 16 + lanes
        pos = plsc.load_gather(cursors, [slot])
        e = base + i * 16 + lanes
        eidv = (e & 1) * (E_ALL // 2) + lax.shift_right_logical(e, 1)
        plsc.store_scatter(eidbuf, [pos], eidv, mask=m)
        plsc.store_scatter(ldstbuf, [pos], local, mask=m)
        plsc.addupdate_scatter(cursors, [slot], ones, mask=m)
        return 0

    lax.fori_loop(0, nvec, place_body, 0)

    pltpu.sync_copy(eidbuf.at[pl.ds(0, REGION)], eid_hbm.at[pl.ds(wid * REGION, REGION)])
    pltpu.sync_copy(ldstbuf.at[pl.ds(0, REGION)], ldst_hbm.at[pl.ds(wid * REGION, REGION)])
    pltpu.sync_copy(startsbuf, tab_hbm.at[pl.ds(wid * 64, 64)])


def _sc_partition(idx_all):
    k = pl.kernel(
        _partition_body,
        out_type=(
            jax.ShapeDtypeStruct((NW * REGION,), jnp.int32),
            jax.ShapeDtypeStruct((NW * REGION,), jnp.int32),
            jax.ShapeDtypeStruct((NW * 64,), jnp.int32),
        ),
        mesh=plsc.VectorSubcoreMesh(core_axis_name="c", subcore_axis_name="s"),
        compiler_params=pltpu.CompilerParams(needs_layout_passes=False),
        scratch_types=[
            pltpu.VMEM((GPW + 16,), jnp.int32),
            pltpu.VMEM((REGION + 16,), jnp.int32),
            pltpu.VMEM((REGION + 16,), jnp.int32),
            pltpu.VMEM((NBUCKET * 16,), jnp.int32),
            pltpu.VMEM((NBUCKET * 16,), jnp.int32),
            pltpu.VMEM((64,), jnp.int32),
        ],
    )
    return k(idx_all)


CH = 256


def _accum_body(msgs_hbm, eid_hbm, ldst_hbm, tab_hbm, out_hbm,
                acc, rows0, rows1, ebuf0, ebuf1, lbuf0, lbuf1, tabbuf,
                sg0, sg1, se0, se1):
    c = lax.axis_index("c")
    s = lax.axis_index("s")
    b = s * 2 + c

    neg_inf = jnp.full((16,), _NEG_INF, jnp.float32)
    lanes = lax.iota(jnp.int32, 16)

    def init_body(i, _):
        for f in range(8):
            acc[i, pl.ds(f * 16, 16)] = neg_inf
        return 0

    lax.fori_loop(0, OUT_STRIDE, init_body, 0)

    pltpu.sync_copy(tab_hbm, tabbuf)

    def st_of(w):
        wv = jnp.minimum(jnp.full((16,), w, jnp.int32), 31)
        return pl.multiple_of(jnp.max(plsc.load_gather(tabbuf, [wv * 64 + b])), 8)

    def en_of(w):
        wv = jnp.minimum(jnp.full((16,), w, jnp.int32), 31)
        return jnp.max(plsc.load_gather(tabbuf, [wv * 64 + 32 + b]))

    def count_body(w, t):
        return t + lax.shift_right_logical(
            jnp.maximum(en_of(w) - st_of(w), 0) + CH - 1, 8)

    nchunks = lax.fori_loop(0, NW, count_body, jnp.int32(0))

    def skip_empty(state):
        def cond(st_):
            w_, c_ = st_
            return jnp.logical_and(c_ >= en_of(w_), w_ < NW - 1)

        def body(st_):
            w_, _ = st_
            return (w_ + 1, st_of(w_ + 1))

        return lax.while_loop(cond, body, state)

    def advance(state):
        w_, c_ = state
        return skip_empty((w_, c_ + CH))

    def fire_eid(state, ebuf, lbuf, sem):
        w_, c_ = state
        off = pl.multiple_of(jnp.minimum(w_, 31) * REGION + c_, 8)
        cp1 = pltpu.make_async_copy(eid_hbm.at[pl.ds(off, CH)], ebuf, sem)
        cp2 = pltpu.make_async_copy(ldst_hbm.at[pl.ds(off, CH)], lbuf, sem)
        cp1.start()
        cp2.start()

    def wait_eid(ebuf, lbuf, sem):
        pltpu.make_async_copy(eid_hbm.at[pl.ds(0, CH)], ebuf, sem).wait()
        pltpu.make_async_copy(ldst_hbm.at[pl.ds(0, CH)], lbuf, sem).wait()

    def fire_gather(ebuf, rows, sem):
        pltpu.make_async_copy(msgs_hbm.at[ebuf.at[pl.ds(0, 128)]],
                              rows.at[pl.ds(0, 128)], sem).start()
        pltpu.make_async_copy(msgs_hbm.at[ebuf.at[pl.ds(128, 128)]],
                              rows.at[pl.ds(128, 128)], sem).start()

    def wait_gather(ebuf, rows, sem):
        pltpu.make_async_copy(msgs_hbm.at[ebuf.at[pl.ds(0, 128)]],
                              rows.at[pl.ds(0, 128)], sem).wait()
        pltpu.make_async_copy(msgs_hbm.at[ebuf.at[pl.ds(128, 128)]],
                              rows.at[pl.ds(128, 128)], sem).wait()

    state0 = skip_empty((jnp.int32(0), st_of(jnp.int32(0))))

    @pl.when(nchunks > 0)
    def _prologue():
        fire_eid(state0, ebuf0, lbuf0, se0)
        wait_eid(ebuf0, lbuf0, se0)
        fire_gather(ebuf0, rows0, sg0)
        state1 = advance(state0)

        @pl.when(nchunks > 1)
        def _():
            fire_eid(state1, ebuf1, lbuf1, se1)

    state1 = advance(state0)

    def chunk_loop(j, carry):
        wj, cj, wn, cn = carry
        p = j & 1

        def even_path():
            @pl.when(j + 1 < nchunks)
            def _():
                wait_eid(ebuf1, lbuf1, se1)
                fire_gather(ebuf1, rows1, sg1)

            wait_gather(ebuf0, rows0, sg0)
            _accum_chunk(acc, rows0, lbuf0, wj, cj, lanes)

            @pl.when(j + 2 < nchunks)
            def _():
                fire_eid((wn2, cn2), ebuf0, lbuf0, se0)

        def odd_path():
            @pl.when(j + 1 < nchunks)
            def _():
                wait_eid(ebuf0, lbuf0, se0)
                fire_gather(ebuf0, rows0, sg0)

            wait_gather(ebuf1, rows1, sg1)
            _accum_chunk(acc, rows1, lbuf1, wj, cj, lanes)

            @pl.when(j + 2 < nchunks)
            def _():
                fire_eid((wn2, cn2), ebuf1, lbuf1, se1)

        wn2, cn2 = advance((wn, cn))

        @pl.when(p == 0)
        def _():
            even_path()

        @pl.when(p == 1)
        def _():
            odd_path()

        return (wn, cn, wn2, cn2)

    def _accum_chunk(acc_, rows_, lbuf_, wj, cj, lanes_):
        clen = jnp.minimum(jnp.int32(CH), en_of(wj) - cj)
        ngroups = lax.shift_right_logical(clen + 15, 4)

        def group_body(g, _):
            rowbase = g * 16
            dvec = lbuf_[pl.ds(rowbase, 16)]
            dsafe = jnp.where(rowbase + lanes_ < clen, dvec,
                              jnp.full((16,), BUCKET_SZ, jnp.int32))
            for l in range(16):
                d = dsafe[l]
                r = rowbase + l
                vals = []
                for f in range(8):
                    cur = acc_[d, pl.ds(f * 16, 16)]
                    val = rows_[r, pl.ds(f * 16, 16)]
                    vals.append(jnp.maximum(cur, val))
                for f in range(8):
                    acc_[d, pl.ds(f * 16, 16)] = vals[f]
            return 0

        lax.fori_loop(0, ngroups, group_body, 0)

    lax.fori_loop(0, nchunks,
                  chunk_loop,
                  (state0[0], state0[1], state1[0], state1[1]))

    pltpu.sync_copy(acc, out_hbm.at[pl.ds(b * OUT_STRIDE, OUT_STRIDE)])


def _sc_accumulate(msgs, eid, ldst, tab):
    k = pl.kernel(
        _accum_body,
        out_type=jax.ShapeDtypeStruct((NW * OUT_STRIDE, D), jnp.float32),
        mesh=plsc.VectorSubcoreMesh(core_axis_name="c", subcore_axis_name="s"),
        compiler_params=pltpu.CompilerParams(needs_layout_passes=False),
        scratch_types=[
            pltpu.VMEM((OUT_STRIDE, D), jnp.float32),
            pltpu.VMEM((CH, D), jnp.float32),
            pltpu.VMEM((CH, D), jnp.float32),
            pltpu.VMEM((CH,), jnp.int32),
            pltpu.VMEM((CH,), jnp.int32),
            pltpu.VMEM((CH,), jnp.int32),
            pltpu.VMEM((CH,), jnp.int32),
            pltpu.VMEM((NW * 64,), jnp.int32),
            pltpu.SemaphoreType.DMA,
            pltpu.SemaphoreType.DMA,
            pltpu.SemaphoreType.DMA,
            pltpu.SemaphoreType.DMA,
        ],
    )
    return k(msgs, eid, ldst, tab)


def _mlp_body(x_ref, wi_ref, bi_ref, wo_ref, bo_ref, o_ref):
    x3 = x_ref[...]
    xe = x3[0]
    xo = x3[1]
    wi = wi_ref[0]
    wo = wo_ref[0]
    h = _mish(jnp.dot(xe.astype(jnp.bfloat16), wi[:D],
                      preferred_element_type=jnp.float32)
              + jnp.dot(xo.astype(jnp.bfloat16), wi[D:],
                        preferred_element_type=jnp.float32)
              + bi_ref[0, 0])
    hb = h.astype(jnp.bfloat16)
    bo = bo_ref[0, 0]
    o_ref[0] = xe + jnp.dot(hb, wo[:, :D], preferred_element_type=jnp.float32) + bo[:D]
    o_ref[1] = xo + jnp.dot(hb, wo[:, D:], preferred_element_type=jnp.float32) + bo[D:]


def _fused_relation_mlp(x3, wi2, bi2, wo2, bo2):
    rows = x3.shape[1]
    n_bin_blocks = (E_BIN // 2) // MLP_BLOCK

    def wsel(i):
        return (jnp.where(i < n_bin_blocks, 0, 1), 0, 0)

    return pl.pallas_call(
        _mlp_body,
        grid=(rows // MLP_BLOCK,),
        in_specs=[
            pl.BlockSpec((2, MLP_BLOCK, D), lambda i: (0, i, 0)),
            pl.BlockSpec((1, 2 * D, 2 * D), wsel),
            pl.BlockSpec((1, 1, 2 * D), wsel),
            pl.BlockSpec((1, 2 * D, 2 * D), wsel),
            pl.BlockSpec((1, 1, 2 * D), wsel),
        ],
        out_specs=pl.BlockSpec((2, MLP_BLOCK, D), lambda i: (0, i, 0)),
        out_shape=jax.ShapeDtypeStruct((2, rows, D), jnp.float32),
    )(x3, wi2, bi2, wo2, bo2)


def _update_body(m_ref, e_ref, w1_ref, w2_ref, bi_ref, wo_ref, bo_ref, o_ref):
    h = (jnp.dot(m_ref[...].astype(jnp.bfloat16), w1_ref[...],
                 preferred_element_type=jnp.float32)
         + jnp.dot(e_ref[...].astype(jnp.bfloat16), w2_ref[...],
                   preferred_element_type=jnp.float32)
         + bi_ref[...])
    o_ref[...] = jnp.dot(_mish(h).astype(jnp.bfloat16), wo_ref[...],
                         preferred_element_type=jnp.float32) + bo_ref[...]


def _update_mlp(max_msg, emb, W_in_up, b_in_up, W_out_up, b_out_up):
    blk = 2000
    w1 = W_in_up[:D].astype(jnp.bfloat16)
    w2 = W_in_up[D:].astype(jnp.bfloat16)
    W_out_up = W_out_up.astype(jnp.bfloat16)
    return pl.pallas_call(
        _update_body,
        grid=(N_NODES // blk,),
        in_specs=[
            pl.BlockSpec((blk, D), lambda i: (i, 0)),
            pl.BlockSpec((blk, D), lambda i: (i, 0)),
            pl.BlockSpec((D, 2 * D), lambda i: (0, 0)),
            pl.BlockSpec((D, 2 * D), lambda i: (0, 0)),
            pl.BlockSpec((2 * D,), lambda i: (0,)),
            pl.BlockSpec((2 * D, D), lambda i: (0, 0)),
            pl.BlockSpec((D,), lambda i: (0,)),
        ],
        out_specs=pl.BlockSpec((blk, D), lambda i: (i, 0)),
        out_shape=jax.ShapeDtypeStruct((N_NODES, D), jnp.float32),
    )(max_msg, emb, w1, w2, b_in_up, W_out_up, b_out_up)


def kernel(node_embeddings, rel_binary, rel_unary,
           W_in_b, b_in_b, W_out_b, b_out_b,
           W_in_u, b_in_u, W_out_u, b_out_u,
           W_in_up, b_in_up, W_out_up, b_out_up):
    idx_all = jnp.concatenate([rel_binary, rel_unary])

    gathered = _sc_gather(idx_all, node_embeddings)

    z = jnp.zeros((D, D), jnp.float32)
    wi_u2 = jnp.block([[W_in_u, z], [z, W_in_u]])
    wo_u2 = jnp.block([[W_out_u, z], [z, W_out_u]])
    wi2 = jnp.stack([W_in_b, wi_u2]).astype(jnp.bfloat16)
    wo2 = jnp.stack([W_out_b, wo_u2]).astype(jnp.bfloat16)
    bi2 = jnp.stack([b_in_b, jnp.concatenate([b_in_u, b_in_u])])[:, None, :]
    bo2 = jnp.stack([b_out_b, jnp.concatenate([b_out_u, b_out_u])])[:, None, :]
    msgs = _fused_relation_mlp(gathered, wi2, bi2, wo2, bo2).reshape(E_ALL, D)

    eid, ldst, tab = _sc_partition(idx_all)
    maxm_padded = _sc_accumulate(msgs, eid, ldst, tab)
    max_msg = maxm_padded.reshape(NW, OUT_STRIDE, D)[:, :BUCKET_SZ].reshape(-1, D)[:N_NODES]

    return _update_mlp(max_msg, node_embeddings, W_in_up, b_in_up, W_out_up, b_out_up)


# speedup vs baseline: 4.6867x; 1.0007x over previous
import jax
import jax.numpy as jnp
from jax import lax
from jax.experimental import pallas as pl
from jax.experimental.pallas import tpu as pltpu
from jax.experimental.pallas import tpu_sc as plsc

N_NODES = 10000
D = 128
E_BIN = 640000
E_UNA = 160000
E_ALL = E_BIN + E_UNA

MLP_BLOCK = 4000

NW = 32
GPW = E_ALL // NW
IDX_CHUNK = 1000
SUB = 128
SUB_TAIL = IDX_CHUNK - 7 * SUB

NBUCKET = 32
BUCKET_SZ = 313
BUCKET_MUL = 53602
REGION = 25512
OUT_STRIDE = 320

_NEG_INF = float("-inf")


def _mish(x):
    sp = jnp.maximum(x, 0.0) + jnp.log1p(jnp.exp(-jnp.abs(x)))
    return x * jnp.tanh(sp)


HPW = E_ALL // 2 // 16


def _gather_body(idx_all_hbm, table_hbm, out_hbm, stage, idxbuf, rows, sem):
    c = lax.axis_index("c")
    s = lax.axis_index("s")
    wid = s * 2 + c
    lanes = lax.iota(jnp.int32, 16)

    def run_half(par, half, base):
        def chunk_body(i, _):
            off = base + i * IDX_CHUNK
            pltpu.sync_copy(idx_all_hbm.at[pl.ds(2 * off, 2 * IDX_CHUNK)],
                            stage.at[pl.ds(0, 2 * IDX_CHUNK)])

            def deint_body(k, _):
                v = plsc.load_gather(stage, [k * 32 + lanes * 2 + par])
                idxbuf[pl.ds(k * 16, 16)] = v
                return 0

            lax.fori_loop(0, (IDX_CHUNK + 15) // 16, deint_body, 0)

            def sub_body(m, _):
                pltpu.async_copy(table_hbm.at[idxbuf.at[pl.ds(m * SUB, SUB)]],
                                 rows, sem).wait()
                pltpu.sync_copy(rows, out_hbm.at[half].at[pl.ds(off + m * SUB, SUB)])
                return 0

            lax.fori_loop(0, 7, sub_body, 0)
            pltpu.async_copy(table_hbm.at[idxbuf.at[pl.ds(7 * SUB, SUB_TAIL)]],
                             rows.at[pl.ds(0, SUB_TAIL)], sem).wait()
            pltpu.sync_copy(rows.at[pl.ds(0, SUB_TAIL)],
                            out_hbm.at[half].at[pl.ds(off + 7 * SUB, SUB_TAIL)])
            return 0

        lax.fori_loop(0, HPW // IDX_CHUNK, chunk_body, 0)

    @pl.when(wid < 16)
    def _():
        run_half(0, 0, wid * HPW)

    @pl.when(wid >= 16)
    def _():
        run_half(1, 1, (wid - 16) * HPW)


def _sc_gather(idx_all, table):
    k = pl.kernel(
        _gather_body,
        out_type=jax.ShapeDtypeStruct((2, E_ALL // 2, D), jnp.float32),
        mesh=plsc.VectorSubcoreMesh(core_axis_name="c", subcore_axis_name="s"),
        compiler_params=pltpu.CompilerParams(needs_layout_passes=False),
        scratch_types=[
            pltpu.VMEM((2 * IDX_CHUNK + 16,), jnp.int32),
            pltpu.VMEM((IDX_CHUNK + 16,), jnp.int32),
            pltpu.VMEM((SUB, D), jnp.float32),
            pltpu.SemaphoreType.DMA,
        ],
    )
    return k(idx_all, table)


def _partition_body(idx_hbm, eid_hbm, ldst_hbm, tab_hbm,
                    idxs, eidbuf, ldstbuf, counters, cursors, startsbuf):
    c = lax.axis_index("c")
    s = lax.axis_index("s")
    wid = s * 2 + c
    base = wid * GPW

    lanes = lax.iota(jnp.int32, 16)
    zeros = jnp.zeros((16,), jnp.int32)
    ones = jnp.ones((16,), jnp.int32)

    pltpu.sync_copy(idx_hbm.at[pl.ds(base, GPW)], idxs.at[pl.ds(0, GPW)])

    idxs[pl.ds(1562 * 16, 16)] = jnp.where(lanes < GPW - 1562 * 16,
                                           idxs[pl.ds(1562 * 16, 16)], zeros)

    for b in range(NBUCKET):
        counters[pl.ds(b * 16, 16)] = zeros

    def zero_body(i, _):
        eidbuf[pl.ds(i * 16, 16)] = zeros
        ldstbuf[pl.ds(i * 16, 16)] = zeros
        return 0

    lax.fori_loop(0, REGION // 16 + 1, zero_body, 0)

    nvec = 1563

    def count_body(i, _):
        m = (i * 16 + lanes) < GPW
        v = idxs[pl.ds(i * 16, 16)]
        bucket = lax.shift_right_logical(v * BUCKET_MUL, 24)
        slot = bucket * 16 + lanes
        plsc.addupdate_scatter(counters, [slot], ones, mask=m)
        return 0

    lax.fori_loop(0, nvec, count_body, 0)

    carry = jnp.int32(0)
    starts = [jnp.int32(0)] * NBUCKET
    ends = [jnp.int32(0)] * NBUCKET
    for b in range(NBUCKET):
        cnts = counters[pl.ds(b * 16, 16)]
        start_al = lax.shift_left(lax.shift_right_logical(carry + 7, 3), 3)
        excl = plsc.cumsum(cnts) - cnts
        cursors[pl.ds(b * 16, 16)] = excl + start_al
        total = jnp.sum(cnts)
        starts[b] = start_al
        ends[b] = start_al + total
        carry = start_al + total

    for j in range(4):
        vals = (starts, ends)[j // 2][(j % 2) * 16:(j % 2) * 16 + 16]
        v = zeros
        for l in range(16):
            v = jnp.where(lanes == l, vals[l], v)
        startsbuf[pl.ds(((j // 2) * 2 + (j % 2)) * 16, 16)] = v

    def place_body(i, _):
        m = (i * 16 + lanes) < GPW
        v = idxs[pl.ds(i * 16, 16)]
        bucket = lax.shift_right_logical(v * BUCKET_MUL, 24)
        local = v - bucket * BUCKET_SZ
        slot = bucket * 16 + lanes
        pos = plsc.load_gather(cursors, [slot])
        e = base + i * 16 + lanes
        eidv = (e & 1) * (E_ALL // 2) + lax.shift_right_logical(e, 1)
        plsc.store_scatter(eidbuf, [pos], eidv, mask=m)
        plsc.store_scatter(ldstbuf, [pos], local, mask=m)
        plsc.addupdate_scatter(cursors, [slot], ones, mask=m)
        return 0

    lax.fori_loop(0, nvec, place_body, 0)

    pltpu.sync_copy(eidbuf.at[pl.ds(0, REGION)], eid_hbm.at[pl.ds(wid * REGION, REGION)])
    pltpu.sync_copy(ldstbuf.at[pl.ds(0, REGION)], ldst_hbm.at[pl.ds(wid * REGION, REGION)])
    pltpu.sync_copy(startsbuf, tab_hbm.at[pl.ds(wid * 64, 64)])


def _sc_partition(idx_all):
    k = pl.kernel(
        _partition_body,
        out_type=(
            jax.ShapeDtypeStruct((NW * REGION,), jnp.int32),
            jax.ShapeDtypeStruct((NW * REGION,), jnp.int32),
            jax.ShapeDtypeStruct((NW * 64,), jnp.int32),
        ),
        mesh=plsc.VectorSubcoreMesh(core_axis_name="c", subcore_axis_name="s"),
        compiler_params=pltpu.CompilerParams(needs_layout_passes=False),
        scratch_types=[
            pltpu.VMEM((GPW + 16,), jnp.int32),
            pltpu.VMEM((REGION + 16,), jnp.int32),
            pltpu.VMEM((REGION + 16,), jnp.int32),
            pltpu.VMEM((NBUCKET * 16,), jnp.int32),
            pltpu.VMEM((NBUCKET * 16,), jnp.int32),
            pltpu.VMEM((64,), jnp.int32),
        ],
    )
    return k(idx_all)


CH = 256


def _accum_body(msgs_hbm, eid_hbm, ldst_hbm, tab_hbm, out_hbm,
                acc, rows0, rows1, ebuf0, ebuf1, lbuf0, lbuf1, tabbuf,
                sg0, sg1, se0, se1):
    c = lax.axis_index("c")
    s = lax.axis_index("s")
    b = s * 2 + c

    neg_inf = jnp.full((16,), _NEG_INF, jnp.float32)
    lanes = lax.iota(jnp.int32, 16)

    def init_body(i, _):
        for f in range(8):
            acc[i, pl.ds(f * 16, 16)] = neg_inf
        return 0

    lax.fori_loop(0, OUT_STRIDE, init_body, 0)

    pltpu.sync_copy(tab_hbm, tabbuf)

    def st_of(w):
        wv = jnp.minimum(jnp.full((16,), w, jnp.int32), 31)
        return pl.multiple_of(jnp.max(plsc.load_gather(tabbuf, [wv * 64 + b])), 8)

    def en_of(w):
        wv = jnp.minimum(jnp.full((16,), w, jnp.int32), 31)
        return jnp.max(plsc.load_gather(tabbuf, [wv * 64 + 32 + b]))

    def count_body(w, t):
        return t + lax.shift_right_logical(
            jnp.maximum(en_of(w) - st_of(w), 0) + CH - 1, 8)

    nchunks = lax.fori_loop(0, NW, count_body, jnp.int32(0))

    def skip_empty(state):
        def cond(st_):
            w_, c_ = st_
            return jnp.logical_and(c_ >= en_of(w_), w_ < NW - 1)

        def body(st_):
            w_, _ = st_
            return (w_ + 1, st_of(w_ + 1))

        return lax.while_loop(cond, body, state)

    def advance(state):
        w_, c_ = state
        return skip_empty((w_, c_ + CH))

    def fire_eid(state, ebuf, lbuf, sem):
        w_, c_ = state
        off = pl.multiple_of(jnp.minimum(w_, 31) * REGION + c_, 8)
        cp1 = pltpu.make_async_copy(eid_hbm.at[pl.ds(off, CH)], ebuf, sem)
        cp2 = pltpu.make_async_copy(ldst_hbm.at[pl.ds(off, CH)], lbuf, sem)
        cp1.start()
        cp2.start()

    def wait_eid(ebuf, lbuf, sem):
        pltpu.make_async_copy(eid_hbm.at[pl.ds(0, CH)], ebuf, sem).wait()
        pltpu.make_async_copy(ldst_hbm.at[pl.ds(0, CH)], lbuf, sem).wait()

    def fire_gather(ebuf, rows, sem):
        pltpu.make_async_copy(msgs_hbm.at[ebuf.at[pl.ds(0, 128)]],
                              rows.at[pl.ds(0, 128)], sem).start()
        pltpu.make_async_copy(msgs_hbm.at[ebuf.at[pl.ds(128, 128)]],
                              rows.at[pl.ds(128, 128)], sem).start()

    def wait_gather(ebuf, rows, sem):
        pltpu.make_async_copy(msgs_hbm.at[ebuf.at[pl.ds(0, 128)]],
                              rows.at[pl.ds(0, 128)], sem).wait()
        pltpu.make_async_copy(msgs_hbm.at[ebuf.at[pl.ds(128, 128)]],
                              rows.at[pl.ds(128, 128)], sem).wait()

    state0 = skip_empty((jnp.int32(0), st_of(jnp.int32(0))))

    @pl.when(nchunks > 0)
    def _prologue():
        fire_eid(state0, ebuf0, lbuf0, se0)
        wait_eid(ebuf0, lbuf0, se0)
        fire_gather(ebuf0, rows0, sg0)
        state1 = advance(state0)

        @pl.when(nchunks > 1)
        def _():
            fire_eid(state1, ebuf1, lbuf1, se1)

    state1 = advance(state0)

    def chunk_loop(j, carry):
        wj, cj, wn, cn = carry
        p = j & 1

        def even_path():
            @pl.when(j + 1 < nchunks)
            def _():
                wait_eid(ebuf1, lbuf1, se1)
                fire_gather(ebuf1, rows1, sg1)

            wait_gather(ebuf0, rows0, sg0)
            _accum_chunk(acc, rows0, lbuf0, wj, cj, lanes)

            @pl.when(j + 2 < nchunks)
            def _():
                fire_eid((wn2, cn2), ebuf0, lbuf0, se0)

        def odd_path():
            @pl.when(j + 1 < nchunks)
            def _():
                wait_eid(ebuf0, lbuf0, se0)
                fire_gather(ebuf0, rows0, sg0)

            wait_gather(ebuf1, rows1, sg1)
            _accum_chunk(acc, rows1, lbuf1, wj, cj, lanes)

            @pl.when(j + 2 < nchunks)
            def _():
                fire_eid((wn2, cn2), ebuf1, lbuf1, se1)

        wn2, cn2 = advance((wn, cn))

        @pl.when(p == 0)
        def _():
            even_path()

        @pl.when(p == 1)
        def _():
            odd_path()

        return (wn, cn, wn2, cn2)

    def _accum_chunk(acc_, rows_, lbuf_, wj, cj, lanes_):
        clen = jnp.minimum(jnp.int32(CH), en_of(wj) - cj)
        ngroups = lax.shift_right_logical(clen + 15, 4)

        def group_body(g, _):
            rowbase = g * 16
            dvec = lbuf_[pl.ds(rowbase, 16)]
            dsafe = jnp.where(rowbase + lanes_ < clen, dvec,
                              jnp.full((16,), BUCKET_SZ, jnp.int32))
            for l in range(16):
                d = dsafe[l]
                r = rowbase + l
                vals = []
                for f in range(8):
                    cur = acc_[d, pl.ds(f * 16, 16)]
                    val = rows_[r, pl.ds(f * 16, 16)]
                    vals.append(jnp.maximum(cur, val))
                for f in range(8):
                    acc_[d, pl.ds(f * 16, 16)] = vals[f]
            return 0

        lax.fori_loop(0, ngroups, group_body, 0)

    lax.fori_loop(0, nchunks,
                  chunk_loop,
                  (state0[0], state0[1], state1[0], state1[1]))

    pltpu.sync_copy(acc, out_hbm.at[pl.ds(b * OUT_STRIDE, OUT_STRIDE)])


def _sc_accumulate(msgs, eid, ldst, tab):
    k = pl.kernel(
        _accum_body,
        out_type=jax.ShapeDtypeStruct((NW * OUT_STRIDE, D), jnp.float32),
        mesh=plsc.VectorSubcoreMesh(core_axis_name="c", subcore_axis_name="s"),
        compiler_params=pltpu.CompilerParams(needs_layout_passes=False),
        scratch_types=[
            pltpu.VMEM((OUT_STRIDE, D), jnp.float32),
            pltpu.VMEM((CH, D), jnp.float32),
            pltpu.VMEM((CH, D), jnp.float32),
            pltpu.VMEM((CH,), jnp.int32),
            pltpu.VMEM((CH,), jnp.int32),
            pltpu.VMEM((CH,), jnp.int32),
            pltpu.VMEM((CH,), jnp.int32),
            pltpu.VMEM((NW * 64,), jnp.int32),
            pltpu.SemaphoreType.DMA,
            pltpu.SemaphoreType.DMA,
            pltpu.SemaphoreType.DMA,
            pltpu.SemaphoreType.DMA,
        ],
    )
    return k(msgs, eid, ldst, tab)


def _mlp_body(x_ref, wi_ref, bi_ref, wo_ref, bo_ref, o_ref):
    x3 = x_ref[...]
    xe = x3[0]
    xo = x3[1]
    wi = wi_ref[0]
    wo = wo_ref[0]
    h = _mish(jnp.dot(xe.astype(jnp.bfloat16), wi[:D],
                      preferred_element_type=jnp.float32)
              + jnp.dot(xo.astype(jnp.bfloat16), wi[D:],
                        preferred_element_type=jnp.float32)
              + bi_ref[0, 0])
    hb = h.astype(jnp.bfloat16)
    bo = bo_ref[0, 0]
    o_ref[0] = xe + jnp.dot(hb, wo[:, :D], preferred_element_type=jnp.float32) + bo[:D]
    o_ref[1] = xo + jnp.dot(hb, wo[:, D:], preferred_element_type=jnp.float32) + bo[D:]


def _fused_relation_mlp(x3, wi2, bi2, wo2, bo2):
    rows = x3.shape[1]
    n_bin_blocks = (E_BIN // 2) // MLP_BLOCK

    def wsel(i):
        return (jnp.where(i < n_bin_blocks, 0, 1), 0, 0)

    return pl.pallas_call(
        _mlp_body,
        grid=(rows // MLP_BLOCK,),
        in_specs=[
            pl.BlockSpec((2, MLP_BLOCK, D), lambda i: (0, i, 0)),
            pl.BlockSpec((1, 2 * D, 2 * D), wsel),
            pl.BlockSpec((1, 1, 2 * D), wsel),
            pl.BlockSpec((1, 2 * D, 2 * D), wsel),
            pl.BlockSpec((1, 1, 2 * D), wsel),
        ],
        out_specs=pl.BlockSpec((2, MLP_BLOCK, D), lambda i: (0, i, 0)),
        out_shape=jax.ShapeDtypeStruct((2, rows, D), jnp.float32),
    )(x3, wi2, bi2, wo2, bo2)


def _update_body(m_ref, e_ref, w1_ref, w2_ref, bi_ref, wo_ref, bo_ref, o_ref):
    h = (jnp.dot(m_ref[...].astype(jnp.bfloat16), w1_ref[...],
                 preferred_element_type=jnp.float32)
         + jnp.dot(e_ref[...].astype(jnp.bfloat16), w2_ref[...],
                   preferred_element_type=jnp.float32)
         + bi_ref[...])
    o_ref[...] = jnp.dot(_mish(h).astype(jnp.bfloat16), wo_ref[...],
                         preferred_element_type=jnp.float32) + bo_ref[...]


def _update_mlp(max_msg, emb, W_in_up, b_in_up, W_out_up, b_out_up):
    blk = 2000
    w1 = W_in_up[:D].astype(jnp.bfloat16)
    w2 = W_in_up[D:].astype(jnp.bfloat16)
    W_out_up = W_out_up.astype(jnp.bfloat16)
    return pl.pallas_call(
        _update_body,
        grid=(N_NODES // blk,),
        in_specs=[
            pl.BlockSpec((blk, D), lambda i: (i, 0)),
            pl.BlockSpec((blk, D), lambda i: (i, 0)),
            pl.BlockSpec((D, 2 * D), lambda i: (0, 0)),
            pl.BlockSpec((D, 2 * D), lambda i: (0, 0)),
            pl.BlockSpec((2 * D,), lambda i: (0,)),
            pl.BlockSpec((2 * D, D), lambda i: (0, 0)),
            pl.BlockSpec((D,), lambda i: (0,)),
        ],
        out_specs=pl.BlockSpec((blk, D), lambda i: (i, 0)),
        out_shape=jax.ShapeDtypeStruct((N_NODES, D), jnp.float32),
    )(max_msg, emb, w1, w2, b_in_up, W_out_up, b_out_up)


def kernel(node_embeddings, rel_binary, rel_unary,
           W_in_b, b_in_b, W_out_b, b_out_b,
           W_in_u, b_in_u, W_out_u, b_out_u,
           W_in_up, b_in_up, W_out_up, b_out_up):
    idx_all = jnp.concatenate([rel_binary, rel_unary])

    gathered = _sc_gather(idx_all, node_embeddings)

    z = jnp.zeros((D, D), jnp.float32)
    wi_u2 = jnp.block([[W_in_u, z], [z, W_in_u]])
    wo_u2 = jnp.block([[W_out_u, z], [z, W_out_u]])
    wi2 = jnp.stack([W_in_b, wi_u2]).astype(jnp.bfloat16)
    wo2 = jnp.stack([W_out_b, wo_u2]).astype(jnp.bfloat16)
    bi2 = jnp.stack([b_in_b, jnp.concatenate([b_in_u, b_in_u])])[:, None, :]
    bo2 = jnp.stack([b_out_b, jnp.concatenate([b_out_u, b_out_u])])[:, None, :]
    msgs = _fused_relation_mlp(gathered, wi2, bi2, wo2, bo2).reshape(E_ALL, D)

    eid, ldst, tab = _sc_partition(idx_all)
    maxm_padded = _sc_accumulate(msgs, eid, ldst, tab)
    max_msg = maxm_padded.reshape(NW, OUT_STRIDE, D)[:, :BUCKET_SZ].reshape(-1, D)[:N_NODES]

    return _update_mlp(max_msg, node_embeddings, W_in_up, b_in_up, W_out_up, b_out_up)
